# Initial kernel scaffold; baseline (speedup 1.0000x reference)
import jax
import jax.numpy as jnp
from jax.experimental import pallas as pl


def kernel(x, latent_vector, style_vector, edge_index, edge_attr, batch_size, nroi, W_fc1, b_fc1, Ws_fc1, bs_fc1, W_fc2, b_fc2, Ws_fc2, bs_fc2, W_fc3, b_fc3, Ws_fc3, bs_fc3, W_fc4, b_fc4, Ws_fc4, bs_fc4, W0_g1, W1_g1, b_g1, W0_g2, W1_g2, b_g2):
    raise NotImplementedError("write your pallas kernel here")


# speedup vs baseline: 6.0701x; 6.0701x over previous
import functools

import jax
import jax.numpy as jnp
from jax import lax
from jax.experimental import pallas as pl
from jax.experimental.pallas import tpu as pltpu
from jax.experimental.pallas import tpu_sc as plsc

F32 = jnp.float32
_HIGH = lax.Precision.HIGHEST


def _dot(a, b):
    return jnp.dot(a, b, preferred_element_type=F32, precision=_HIGH)


def _lrelu(x):
    return jnp.where(x >= 0, x, 0.2 * x)


def _a_body(x_ref, lat_ref, sty_ref, degp_ref,
            W1_ref, b1_ref, Ws1_ref, bs1_ref,
            W2_ref, b2_ref, Ws2_ref, bs2_ref,
            W3_ref, b3_ref, Ws3_ref, bs3_ref,
            W4_ref, b4_ref, Ws4_ref, bs4_ref,
            W0g_ref, W1g_ref, bg_ref,
            y1_ref, z1_ref, dinv_ref):
    xb = x_ref[...]
    R = xb.shape[0]
    lv = lat_ref[0]
    h = jnp.concatenate([xb, jnp.broadcast_to(lv, (R, lv.shape[1]))], axis=1)
    sv = sty_ref[0]

    def unit(h, W_r, b_r, Ws_r, bs_r):
        t = _dot(h, W_r[...]) + b_r[...]
        mu = jnp.mean(t, axis=0, keepdims=True)
        d = t - mu
        var = jnp.mean(d * d, axis=0, keepdims=True)
        s = _dot(sv, Ws_r[...]) + bs_r[...]
        F = t.shape[1]
        gamma = s[:, :F]
        beta = s[:, F:]
        out = d * lax.rsqrt(var + 1e-5) * (1.0 + gamma) + beta
        return _lrelu(out)

    h = unit(h, W1_ref, b1_ref, Ws1_ref, bs1_ref)
    h = unit(h, W2_ref, b2_ref, Ws2_ref, bs2_ref)
    h = unit(h, W3_ref, b3_ref, Ws3_ref, bs3_ref)
    h = unit(h, W4_ref, b4_ref, Ws4_ref, bs4_ref)

    y1 = _dot(h, W1g_ref[...])
    for cc in range(y1.shape[1] // 128):
        y1_ref[cc] = y1[:, cc * 128:(cc + 1) * 128]
    z1_ref[...] = _dot(h, W0g_ref[...]) + bg_ref[...]

    degb = degp_ref[0, 0] + degp_ref[1, 0]
    dinv_ref[0] = jnp.where(degb > 0, 1.0 / jnp.sqrt(degb + 1e-12), 0.0)


def _run_a(x, lat, sty, degp, Ws, F2):
    N, IN = x.shape
    B, LAT = lat.shape
    R = N // B
    S = sty.shape[1]
    NC = F2 // 128
    full = lambda shape: pl.BlockSpec(shape, lambda b: tuple(0 for _ in shape))
    in_specs = [
        pl.BlockSpec((R, IN), lambda b: (b, 0)),
        pl.BlockSpec((1, 1, LAT), lambda b: (b, 0, 0)),
        pl.BlockSpec((1, 1, S), lambda b: (b, 0, 0)),
        pl.BlockSpec((2, 1, 1, R), lambda b: (0, b, 0, 0)),
    ] + [full(w.shape) for w in Ws]
    out_shapes = [
        jax.ShapeDtypeStruct((NC, N, 128), F32),
        jax.ShapeDtypeStruct((N, F2), F32),
        jax.ShapeDtypeStruct((B, 1, R), F32),
    ]
    out_specs = [
        pl.BlockSpec((NC, R, 128), lambda b: (0, b, 0)),
        pl.BlockSpec((R, F2), lambda b: (b, 0)),
        pl.BlockSpec((1, 1, R), lambda b: (b, 0, 0)),
    ]
    return pl.pallas_call(
        _a_body, grid=(B,), in_specs=in_specs, out_specs=out_specs,
        out_shape=out_shapes,
    )(x, lat.reshape(B, 1, LAT), sty.reshape(B, 1, S),
      degp.reshape(2, B, 1, R), *Ws)


def _lane_bcast(vec, lane):
    idx = jnp.full((16, 1), lane, jnp.int32)
    dnums = lax.GatherDimensionNumbers(
        offset_dims=(), collapsed_slice_dims=(0,), start_index_map=(0,))
    return lax.gather(vec, idx, dnums, (1,),
                      mode=lax.GatherScatterMode.PROMISE_IN_BOUNDS)


def _zero2d(ref, rows, cols):
    z = jnp.zeros((16,), F32)

    def body(r, _):
        for v in range(cols // 16):
            ref[r, pl.ds(v * 16, 16)] = z
        return 0

    lax.fori_loop(0, rows, body, 0)


def _sc_deg(dst, ew, N, E):
    NBLK_SC = E // 256
    per_tile = -(-NBLK_SC // 16)
    mesh = plsc.VectorSubcoreMesh(core_axis_name="c", subcore_axis_name="s")

    @functools.partial(
        pl.kernel, mesh=mesh,
        out_type=jax.ShapeDtypeStruct((2 * N,), F32),
        scratch_types=[
            pltpu.VMEM((1, 128), jnp.int32),
            pltpu.VMEM((128,), F32),
            pltpu.VMEM((N,), F32),
            pltpu.VMEM_SHARED((N,), F32),
        ],
        compiler_params=pltpu.CompilerParams(needs_layout_passes=False),
    )
    def deg_kernel(dst_hbm, ew_hbm, out_hbm, idx_v, val_v, zb_v, acc_sh):
        c = lax.axis_index("c")
        s = lax.axis_index("s")
        z = jnp.zeros((16,), F32)

        def zbody(i, _):
            zb_v[pl.ds(i * 16, 16)] = z
            return 0

        lax.fori_loop(0, N // 16, zbody, 0)

        @pl.when(s == 0)
        def _():
            pltpu.sync_copy(zb_v, acc_sh)

        plsc.subcore_barrier()

        def step(i, _):
            blk = c * NBLK_SC + s + 16 * i

            @pl.when(s + 16 * i < NBLK_SC)
            def _():
                base = blk * 128
                pltpu.sync_copy(dst_hbm.at[pl.ds(base, 128)], idx_v.at[0])
                pltpu.sync_copy(ew_hbm.at[pl.ds(base, 128)], val_v)
                pltpu.sync_copy(val_v, acc_sh.at[idx_v.at[0]], add=True)
            return 0

        lax.fori_loop(0, per_tile, step, 0)
        plsc.subcore_barrier()

        @pl.when(s == 0)
        def _():
            pltpu.sync_copy(acc_sh, zb_v)
            pltpu.sync_copy(zb_v, out_hbm.at[pl.ds(c * N, N)])

    return deg_kernel(dst, ew)


def _sc_edge_scatter(yflat, src, dst, ew, dinv, N, E, NC):
    NBLK_SC = E // 256
    per_tile = -(-NBLK_SC // 16)
    Npad = -(-N // 2048) * 2048
    RPT = Npad // 16
    mesh = plsc.VectorSubcoreMesh(core_axis_name="c", subcore_axis_name="s")

    @functools.partial(
        pl.kernel, mesh=mesh,
        out_type=jax.ShapeDtypeStruct((2 * NC * Npad, 128), F32),
        scratch_types=[
            pltpu.VMEM((N,), F32),
            pltpu.VMEM((1, 128), jnp.int32),
            pltpu.VMEM((1, 128), jnp.int32),
            pltpu.VMEM((1, 128), jnp.int32),
            pltpu.VMEM((128,), F32),
            pltpu.VMEM((128,), F32),
            pltpu.VMEM((128, 128), F32),
            pltpu.VMEM((128, 128), F32),
            pltpu.VMEM_SHARED((Npad, 128), F32),
            pltpu.SemaphoreType.DMA,
        ],
        compiler_params=pltpu.CompilerParams(needs_layout_passes=False),
    )
    def scat_kernel(y_hbm, src_hbm, dst_hbm, ew_hbm, dinv_hbm, out_hbm,
                    dinv_v, src2_v, dst2_v, adj_v, ew_v, norm_v, rbuf,
                    zbuf, acc_sh, gsem):
        k_core = lax.axis_index("c")
        s = lax.axis_index("s")
        _zero2d(zbuf, 128, 128)
        pltpu.sync_copy(dinv_hbm, dinv_v)

        for c in range(NC):
            for j in range(RPT // 128):
                pltpu.sync_copy(zbuf,
                                acc_sh.at[pl.ds(s * RPT + j * 128, 128)])
            plsc.subcore_barrier()

            def step(i, _):
                lblk = s + 16 * i

                @pl.when(lblk < NBLK_SC)
                def _():
                    base = (k_core * NBLK_SC + lblk) * 128
                    pltpu.sync_copy(src_hbm.at[pl.ds(base, 128)],
                                    src2_v.at[0])
                    pltpu.sync_copy(dst_hbm.at[pl.ds(base, 128)],
                                    dst2_v.at[0])
                    pltpu.sync_copy(ew_hbm.at[pl.ds(base, 128)], ew_v)
                    for k in range(8):
                        sl = pl.ds(k * 16, 16)
                        sv = src2_v[0, sl]
                        dv = dst2_v[0, sl]
                        norm_v[sl] = -(ew_v[sl]
                                       * plsc.load_gather(dinv_v, [sv])
                                       * plsc.load_gather(dinv_v, [dv]))
                        adj_v[0, sl] = sv + c * N
                    pltpu.async_copy(y_hbm.at[adj_v.at[0]], rbuf, gsem).wait()

                    for k in range(8):
                        nv16 = norm_v[pl.ds(k * 16, 16)]

                        def scale(lane, _):
                            nrm = _lane_bcast(nv16, lane)
                            e = k * 16 + lane
                            for v in range(8):
                                cs = pl.ds(v * 16, 16)
                                rbuf[e, cs] = rbuf[e, cs] * nrm
                            return 0

                        lax.fori_loop(0, 16, scale, 0)
                    pltpu.sync_copy(rbuf, acc_sh.at[dst2_v.at[0]], add=True)
                return 0

            lax.fori_loop(0, per_tile, step, 0)
            plsc.subcore_barrier()
            for j in range(RPT // 128):
                r0 = s * RPT + j * 128
                pltpu.sync_copy(acc_sh.at[pl.ds(r0, 128)], rbuf)
                pltpu.sync_copy(
                    rbuf, out_hbm.at[pl.ds((k_core * NC + c) * Npad + r0,
                                           128)])

    return scat_kernel(yflat, src, dst, ew, dinv)


def _c1_body(z_ref, t_ref, out_ref, stats_ref, acc_ref):
    i = pl.program_id(0)
    nb = pl.num_programs(0)
    t = t_ref[...]
    NC = t.shape[1]
    o = z_ref[...] + jnp.concatenate(
        [t[0, cc] + t[1, cc] for cc in range(NC)], axis=1)
    out_ref[...] = o

    @pl.when(i == 0)
    def _():
        acc_ref[...] = jnp.zeros_like(acc_ref)

    s1 = jnp.sum(o, axis=0, keepdims=True)
    s2 = jnp.sum(o * o, axis=0, keepdims=True)
    acc_ref[...] += jnp.concatenate([s1, s2], axis=0)

    @pl.when(i == nb - 1)
    def _():
        stats_ref[...] = acc_ref[...]


def _run_c1(z, t, RB):
    N, F = z.shape
    NC = t.shape[1]
    grid = (N // RB,)
    return pl.pallas_call(
        _c1_body, grid=grid,
        in_specs=[
            pl.BlockSpec((RB, F), lambda i: (i, 0)),
            pl.BlockSpec((2, NC, RB, 128), lambda i: (0, 0, i, 0)),
        ],
        out_specs=[
            pl.BlockSpec((RB, F), lambda i: (i, 0)),
            pl.BlockSpec((2, F), lambda i: (0, 0)),
        ],
        out_shape=[
            jax.ShapeDtypeStruct((N, F), F32),
            jax.ShapeDtypeStruct((2, F), F32),
        ],
        scratch_shapes=[pltpu.VMEM((2, F), F32)],
    )(z, t)


def _c2_body(o_ref, st_ref, W0_ref, W1_ref, b_ref, y2_ref, z2_ref, *, n):
    st = st_ref[...]
    mu = st[0:1] / n
    var = st[1:2] / n - mu * mu
    h = _lrelu((o_ref[...] - mu) * lax.rsqrt(var + 1e-5))
    y2 = _dot(h, W1_ref[...])
    for cc in range(y2.shape[1] // 128):
        y2_ref[cc] = y2[:, cc * 128:(cc + 1) * 128]
    z2_ref[...] = _dot(h, W0_ref[...]) + b_ref[...]


def _run_c2(out1, stats, W0, W1, b, RB):
    N, F = out1.shape
    O = W0.shape[1]
    NC2 = O // 128
    grid = (N // RB,)
    return pl.pallas_call(
        functools.partial(_c2_body, n=float(N)), grid=grid,
        in_specs=[
            pl.BlockSpec((RB, F), lambda i: (i, 0)),
            pl.BlockSpec((2, F), lambda i: (0, 0)),
            pl.BlockSpec(W0.shape, lambda i: (0, 0)),
            pl.BlockSpec(W1.shape, lambda i: (0, 0)),
            pl.BlockSpec((1, O), lambda i: (0, 0)),
        ],
        out_specs=[
            pl.BlockSpec((NC2, RB, 128), lambda i: (0, i, 0)),
            pl.BlockSpec((RB, O), lambda i: (i, 0)),
        ],
        out_shape=[
            jax.ShapeDtypeStruct((NC2, N, 128), F32),
            jax.ShapeDtypeStruct((N, O), F32),
        ],
    )(out1, stats, W0, W1, b)


def _d2_body(o_ref, st_ref, out_ref, *, n):
    st = st_ref[...]
    mu = st[0:1] / n
    var = st[1:2] / n - mu * mu
    out_ref[...] = _lrelu((o_ref[...] - mu) * lax.rsqrt(var + 1e-5))


def _run_d2(out2, stats, RB):
    N, F = out2.shape
    grid = (N // RB,)
    return pl.pallas_call(
        functools.partial(_d2_body, n=float(N)), grid=grid,
        in_specs=[
            pl.BlockSpec((RB, F), lambda i: (i, 0)),
            pl.BlockSpec((2, F), lambda i: (0, 0)),
        ],
        out_specs=pl.BlockSpec((RB, F), lambda i: (i, 0)),
        out_shape=jax.ShapeDtypeStruct((N, F), F32),
    )(out2, stats)


def kernel(x, latent_vector, style_vector, edge_index, edge_attr, batch_size,
           nroi, W_fc1, b_fc1, Ws_fc1, bs_fc1, W_fc2, b_fc2, Ws_fc2, bs_fc2,
           W_fc3, b_fc3, Ws_fc3, bs_fc3, W_fc4, b_fc4, Ws_fc4, bs_fc4,
           W0_g1, W1_g1, b_g1, W0_g2, W1_g2, b_g2):
    N, IN = x.shape
    B, LAT = latent_vector.shape
    R = N // B
    E = edge_index.shape[1]
    F2 = W_fc4.shape[1]
    O = W0_g2.shape[1]
    src = edge_index[0]
    dst = edge_index[1]

    degp = _sc_deg(dst, edge_attr, N, E).reshape(2, B, R)

    Ws = [W_fc1, b_fc1.reshape(1, -1), Ws_fc1, bs_fc1.reshape(1, -1),
          W_fc2, b_fc2.reshape(1, -1), Ws_fc2, bs_fc2.reshape(1, -1),
          W_fc3, b_fc3.reshape(1, -1), Ws_fc3, bs_fc3.reshape(1, -1),
          W_fc4, b_fc4.reshape(1, -1), Ws_fc4, bs_fc4.reshape(1, -1),
          W0_g1, W1_g1, b_g1.reshape(1, -1)]
    y1, z1, dinv = _run_a(x, latent_vector, style_vector, degp, Ws, F2)
    dinv = dinv.reshape(N)

    Npad = -(-N // 2048) * 2048

    def unpad(t, NC):
        return t.reshape(2, NC, Npad, 128)[:, :, :N]

    NC1 = F2 // 128
    NC2 = O // 128
    t1 = unpad(_sc_edge_scatter(y1.reshape(NC1 * N, 128), src, dst,
                                edge_attr, dinv, N, E, NC1), NC1)
    out1, stats1 = _run_c1(z1, t1, 1000)
    y2, z2 = _run_c2(out1, stats1, W0_g2, W1_g2, b_g2.reshape(1, -1), 1000)

    t2 = unpad(_sc_edge_scatter(y2.reshape(NC2 * N, 128), src, dst,
                                edge_attr, dinv, N, E, NC2), NC2)
    out2, stats2 = _run_c1(z2, t2, 1000)
    h = _run_d2(out2, stats2, 1000)
    return h.reshape(B, R, O)


# speedup vs baseline: 6.4223x; 1.0580x over previous
import functools

import jax
import jax.numpy as jnp
from jax import lax
from jax.experimental import pallas as pl
from jax.experimental.pallas import tpu as pltpu
from jax.experimental.pallas import tpu_sc as plsc

F32 = jnp.float32
_HIGH = lax.Precision.HIGHEST


def _dot(a, b):
    return jnp.dot(a, b, preferred_element_type=F32, precision=_HIGH)


def _lrelu(x):
    return jnp.where(x >= 0, x, 0.2 * x)


def _a_body(x_ref, lat_ref, sty_ref, degp_ref,
            W1_ref, b1_ref, Ws1_ref, bs1_ref,
            W2_ref, b2_ref, Ws2_ref, bs2_ref,
            W3_ref, b3_ref, Ws3_ref, bs3_ref,
            W4_ref, b4_ref, Ws4_ref, bs4_ref,
            W0g_ref, W1g_ref, bg_ref,
            y1_ref, z1_ref, dinv_ref):
    xb = x_ref[...]
    R = xb.shape[0]
    lv = lat_ref[0]
    h = jnp.concatenate([xb, jnp.broadcast_to(lv, (R, lv.shape[1]))], axis=1)
    sv = sty_ref[0]

    def unit(h, W_r, b_r, Ws_r, bs_r):
        t = _dot(h, W_r[...]) + b_r[...]
        mu = jnp.mean(t, axis=0, keepdims=True)
        d = t - mu
        var = jnp.mean(d * d, axis=0, keepdims=True)
        s = _dot(sv, Ws_r[...]) + bs_r[...]
        F = t.shape[1]
        gamma = s[:, :F]
        beta = s[:, F:]
        out = d * lax.rsqrt(var + 1e-5) * (1.0 + gamma) + beta
        return _lrelu(out)

    h = unit(h, W1_ref, b1_ref, Ws1_ref, bs1_ref)
    h = unit(h, W2_ref, b2_ref, Ws2_ref, bs2_ref)
    h = unit(h, W3_ref, b3_ref, Ws3_ref, bs3_ref)
    h = unit(h, W4_ref, b4_ref, Ws4_ref, bs4_ref)

    y1 = _dot(h, W1g_ref[...])
    for cc in range(y1.shape[1] // 64):
        y1_ref[cc] = y1[:, cc * 64:(cc + 1) * 64]
    z1_ref[...] = _dot(h, W0g_ref[...]) + bg_ref[...]

    degb = degp_ref[0, 0] + degp_ref[1, 0]
    dinv_ref[0] = jnp.where(degb > 0, 1.0 / jnp.sqrt(degb + 1e-12), 0.0)


def _run_a(x, lat, sty, degp, Ws, F2):
    N, IN = x.shape
    B, LAT = lat.shape
    R = N // B
    S = sty.shape[1]
    NC = F2 // 64
    full = lambda shape: pl.BlockSpec(shape, lambda b: tuple(0 for _ in shape))
    in_specs = [
        pl.BlockSpec((R, IN), lambda b: (b, 0)),
        pl.BlockSpec((1, 1, LAT), lambda b: (b, 0, 0)),
        pl.BlockSpec((1, 1, S), lambda b: (b, 0, 0)),
        pl.BlockSpec((2, 1, 1, R), lambda b: (0, b, 0, 0)),
    ] + [full(w.shape) for w in Ws]
    out_shapes = [
        jax.ShapeDtypeStruct((NC, N, 64), F32),
        jax.ShapeDtypeStruct((N, F2), F32),
        jax.ShapeDtypeStruct((B, 1, R), F32),
    ]
    out_specs = [
        pl.BlockSpec((NC, R, 64), lambda b: (0, b, 0)),
        pl.BlockSpec((R, F2), lambda b: (b, 0)),
        pl.BlockSpec((1, 1, R), lambda b: (b, 0, 0)),
    ]
    return pl.pallas_call(
        _a_body, grid=(B,), in_specs=in_specs, out_specs=out_specs,
        out_shape=out_shapes,
    )(x, lat.reshape(B, 1, LAT), sty.reshape(B, 1, S),
      degp.reshape(2, B, 1, R), *Ws)


def _lane_bcast(vec, lane):
    idx = jnp.full((16, 1), lane, jnp.int32)
    dnums = lax.GatherDimensionNumbers(
        offset_dims=(), collapsed_slice_dims=(0,), start_index_map=(0,))
    return lax.gather(vec, idx, dnums, (1,),
                      mode=lax.GatherScatterMode.PROMISE_IN_BOUNDS)


def _zero2d(ref, rows, cols):
    z = jnp.zeros((16,), F32)

    def body(r, _):
        for v in range(cols // 16):
            ref[r, pl.ds(v * 16, 16)] = z
        return 0

    lax.fori_loop(0, rows, body, 0)


def _sc_deg(dst, ew, N, E):
    NBLK_SC = E // 256
    per_tile = -(-NBLK_SC // 16)
    mesh = plsc.VectorSubcoreMesh(core_axis_name="c", subcore_axis_name="s")

    @functools.partial(
        pl.kernel, mesh=mesh,
        out_type=jax.ShapeDtypeStruct((2 * N,), F32),
        scratch_types=[
            pltpu.VMEM((1, 128), jnp.int32),
            pltpu.VMEM((128,), F32),
            pltpu.VMEM((N,), F32),
            pltpu.VMEM_SHARED((N,), F32),
        ],
        compiler_params=pltpu.CompilerParams(needs_layout_passes=False),
    )
    def deg_kernel(dst_hbm, ew_hbm, out_hbm, idx_v, val_v, zb_v, acc_sh):
        c = lax.axis_index("c")
        s = lax.axis_index("s")
        z = jnp.zeros((16,), F32)

        def zbody(i, _):
            zb_v[pl.ds(i * 16, 16)] = z
            return 0

        lax.fori_loop(0, N // 16, zbody, 0)

        @pl.when(s == 0)
        def _():
            pltpu.sync_copy(zb_v, acc_sh)

        plsc.subcore_barrier()

        def step(i, _):
            blk = c * NBLK_SC + s + 16 * i

            @pl.when(s + 16 * i < NBLK_SC)
            def _():
                base = blk * 128
                pltpu.sync_copy(dst_hbm.at[pl.ds(base, 128)], idx_v.at[0])
                pltpu.sync_copy(ew_hbm.at[pl.ds(base, 128)], val_v)
                pltpu.sync_copy(val_v, acc_sh.at[idx_v.at[0]], add=True)
            return 0

        lax.fori_loop(0, per_tile, step, 0)
        plsc.subcore_barrier()

        @pl.when(s == 0)
        def _():
            pltpu.sync_copy(acc_sh, zb_v)
            pltpu.sync_copy(zb_v, out_hbm.at[pl.ds(c * N, N)])

    return deg_kernel(dst, ew)


def _sc_edge_scatter(yflat, src, dst, ew, dinv, N, E, NC):
    NBLK_SC = E // 256
    per_tile = -(-NBLK_SC // 16)
    Npad = -(-N // 2048) * 2048
    RPT = Npad // 16
    mesh = plsc.VectorSubcoreMesh(core_axis_name="c", subcore_axis_name="s")

    SB = 4
    SG = -(-per_tile // SB)
    CW = 64

    @functools.partial(
        pl.kernel, mesh=mesh,
        out_type=jax.ShapeDtypeStruct((2 * NC * Npad, CW), F32),
        scratch_types=[
            pltpu.VMEM((N,), F32),
            pltpu.VMEM((SB, 128), jnp.int32),
            pltpu.VMEM((SB, 128), jnp.int32),
            pltpu.VMEM((SB, 128), jnp.int32),
            pltpu.VMEM((SB, 128), F32),
            pltpu.VMEM((SB, 128), F32),
            pltpu.VMEM((SB * 128, CW), F32),
            pltpu.VMEM((128, CW), F32),
            pltpu.VMEM_SHARED((Npad, CW), F32),
            pltpu.SemaphoreType.DMA,
            pltpu.SemaphoreType.DMA,
            pltpu.SemaphoreType.DMA,
        ],
        compiler_params=pltpu.CompilerParams(
            needs_layout_passes=False, use_tc_tiling_on_sc=False),
    )
    def scat_kernel(y_hbm, src_hbm, dst_hbm, ew_hbm, dinv_hbm, out_hbm,
                    dinv_v, src2_v, dst2_v, adj_v, ew_v, norm_v, rbuf,
                    zbuf, acc_sh, esem, gsem, ssem):
        k_core = lax.axis_index("c")
        s = lax.axis_index("s")
        _zero2d(zbuf, 128, CW)
        pltpu.sync_copy(dinv_hbm, dinv_v)
        cnt = lax.div(NBLK_SC - 1 - s, 16) + 1

        for c in range(NC):
            zw = []
            for j in range(RPT // 128):
                zw.append(pltpu.async_copy(
                    zbuf, acc_sh.at[pl.ds(s * RPT + j * 128, 128)], ssem))
            for w in zw:
                w.wait()
            plsc.subcore_barrier()

            def step(g, _):
                stage = []
                bases = []
                for b in range(SB):
                    i_eff = jnp.minimum(g * SB + b, cnt - 1)
                    base = (k_core * NBLK_SC + s + 16 * i_eff) * 128
                    bases.append(base)
                    stage.append(pltpu.async_copy(
                        src_hbm.at[pl.ds(base, 128)], src2_v.at[b], esem))
                    stage.append(pltpu.async_copy(
                        dst_hbm.at[pl.ds(base, 128)], dst2_v.at[b], esem))
                    stage.append(pltpu.async_copy(
                        ew_hbm.at[pl.ds(base, 128)], ew_v.at[b], esem))
                for w in stage:
                    w.wait()
                for b in range(SB):
                    valid = (g * SB + b) < cnt
                    for k in range(8):
                        sl = pl.ds(k * 16, 16)
                        sv = src2_v[b, sl]
                        dv = dst2_v[b, sl]
                        nv = -(ew_v[b, sl]
                               * plsc.load_gather(dinv_v, [sv])
                               * plsc.load_gather(dinv_v, [dv]))
                        norm_v[b, sl] = jnp.where(valid, nv,
                                                  jnp.zeros((16,), F32))
                        adj_v[b, sl] = sv + c * N
                gw = []
                for b in range(SB):
                    gw.append(pltpu.async_copy(
                        y_hbm.at[adj_v.at[b]],
                        rbuf.at[pl.ds(b * 128, 128)], gsem))
                for w in gw:
                    w.wait()
                for b in range(SB):
                    for k in range(8):
                        nv16 = norm_v[b, pl.ds(k * 16, 16)]

                        def scale(lane, _):
                            nrm = _lane_bcast(nv16, lane)
                            e = b * 128 + k * 16 + lane
                            for v in range(CW // 16):
                                cs = pl.ds(v * 16, 16)
                                rbuf[e, cs] = rbuf[e, cs] * nrm
                            return 0

                        lax.fori_loop(0, 16, scale, 0)
                sw = []
                for b in range(SB):
                    sw.append(pltpu.async_copy(
                        rbuf.at[pl.ds(b * 128, 128)],
                        acc_sh.at[dst2_v.at[b]], ssem, add=True))
                for w in sw:
                    w.wait()
                return 0

            lax.fori_loop(0, SG, step, 0)
            plsc.subcore_barrier()
            for j0 in range(0, RPT // 128, SB):
                jn = min(SB, RPT // 128 - j0)
                rw = []
                for j in range(jn):
                    r0 = s * RPT + (j0 + j) * 128
                    rw.append(pltpu.async_copy(
                        acc_sh.at[pl.ds(r0, 128)],
                        rbuf.at[pl.ds(j * 128, 128)], gsem))
                for w in rw:
                    w.wait()
                ww = []
                for j in range(jn):
                    r0 = s * RPT + (j0 + j) * 128
                    ww.append(pltpu.async_copy(
                        rbuf.at[pl.ds(j * 128, 128)],
                        out_hbm.at[pl.ds((k_core * NC + c) * Npad + r0,
                                         128)], gsem))
                for w in ww:
                    w.wait()

    return scat_kernel(yflat, src, dst, ew, dinv)


def _c1_body(z_ref, t_ref, out_ref, stats_ref, acc_ref):
    i = pl.program_id(0)
    nb = pl.num_programs(0)
    t = t_ref[...]
    NC = t.shape[1]
    o = z_ref[...] + jnp.concatenate(
        [t[0, cc] + t[1, cc] for cc in range(NC)], axis=1)
    out_ref[...] = o

    @pl.when(i == 0)
    def _():
        acc_ref[...] = jnp.zeros_like(acc_ref)

    s1 = jnp.sum(o, axis=0, keepdims=True)
    s2 = jnp.sum(o * o, axis=0, keepdims=True)
    acc_ref[...] += jnp.concatenate([s1, s2], axis=0)

    @pl.when(i == nb - 1)
    def _():
        stats_ref[...] = acc_ref[...]


def _run_c1(z, t, RB):
    N, F = z.shape
    NC = t.shape[1]
    CW = t.shape[3]
    grid = (N // RB,)
    return pl.pallas_call(
        _c1_body, grid=grid,
        in_specs=[
            pl.BlockSpec((RB, F), lambda i: (i, 0)),
            pl.BlockSpec((2, NC, RB, CW), lambda i: (0, 0, i, 0)),
        ],
        out_specs=[
            pl.BlockSpec((RB, F), lambda i: (i, 0)),
            pl.BlockSpec((2, F), lambda i: (0, 0)),
        ],
        out_shape=[
            jax.ShapeDtypeStruct((N, F), F32),
            jax.ShapeDtypeStruct((2, F), F32),
        ],
        scratch_shapes=[pltpu.VMEM((2, F), F32)],
    )(z, t)


def _c2_body(o_ref, st_ref, W0_ref, W1_ref, b_ref, y2_ref, z2_ref, *, n):
    st = st_ref[...]
    mu = st[0:1] / n
    var = st[1:2] / n - mu * mu
    h = _lrelu((o_ref[...] - mu) * lax.rsqrt(var + 1e-5))
    y2 = _dot(h, W1_ref[...])
    for cc in range(y2.shape[1] // 64):
        y2_ref[cc] = y2[:, cc * 64:(cc + 1) * 64]
    z2_ref[...] = _dot(h, W0_ref[...]) + b_ref[...]


def _run_c2(out1, stats, W0, W1, b, RB):
    N, F = out1.shape
    O = W0.shape[1]
    NC2 = O // 64
    grid = (N // RB,)
    return pl.pallas_call(
        functools.partial(_c2_body, n=float(N)), grid=grid,
        in_specs=[
            pl.BlockSpec((RB, F), lambda i: (i, 0)),
            pl.BlockSpec((2, F), lambda i: (0, 0)),
            pl.BlockSpec(W0.shape, lambda i: (0, 0)),
            pl.BlockSpec(W1.shape, lambda i: (0, 0)),
            pl.BlockSpec((1, O), lambda i: (0, 0)),
        ],
        out_specs=[
            pl.BlockSpec((NC2, RB, 64), lambda i: (0, i, 0)),
            pl.BlockSpec((RB, O), lambda i: (i, 0)),
        ],
        out_shape=[
            jax.ShapeDtypeStruct((NC2, N, 64), F32),
            jax.ShapeDtypeStruct((N, O), F32),
        ],
    )(out1, stats, W0, W1, b)


def _d2_body(o_ref, st_ref, out_ref, *, n):
    st = st_ref[...]
    mu = st[0:1] / n
    var = st[1:2] / n - mu * mu
    out_ref[...] = _lrelu((o_ref[...] - mu) * lax.rsqrt(var + 1e-5))


def _run_d2(out2, stats, RB):
    N, F = out2.shape
    grid = (N // RB,)
    return pl.pallas_call(
        functools.partial(_d2_body, n=float(N)), grid=grid,
        in_specs=[
            pl.BlockSpec((RB, F), lambda i: (i, 0)),
            pl.BlockSpec((2, F), lambda i: (0, 0)),
        ],
        out_specs=pl.BlockSpec((RB, F), lambda i: (i, 0)),
        out_shape=jax.ShapeDtypeStruct((N, F), F32),
    )(out2, stats)


def kernel(x, latent_vector, style_vector, edge_index, edge_attr, batch_size,
           nroi, W_fc1, b_fc1, Ws_fc1, bs_fc1, W_fc2, b_fc2, Ws_fc2, bs_fc2,
           W_fc3, b_fc3, Ws_fc3, bs_fc3, W_fc4, b_fc4, Ws_fc4, bs_fc4,
           W0_g1, W1_g1, b_g1, W0_g2, W1_g2, b_g2):
    N, IN = x.shape
    B, LAT = latent_vector.shape
    R = N // B
    E = edge_index.shape[1]
    F2 = W_fc4.shape[1]
    O = W0_g2.shape[1]
    src = edge_index[0]
    dst = edge_index[1]

    degp = _sc_deg(dst, edge_attr, N, E).reshape(2, B, R)

    Ws = [W_fc1, b_fc1.reshape(1, -1), Ws_fc1, bs_fc1.reshape(1, -1),
          W_fc2, b_fc2.reshape(1, -1), Ws_fc2, bs_fc2.reshape(1, -1),
          W_fc3, b_fc3.reshape(1, -1), Ws_fc3, bs_fc3.reshape(1, -1),
          W_fc4, b_fc4.reshape(1, -1), Ws_fc4, bs_fc4.reshape(1, -1),
          W0_g1, W1_g1, b_g1.reshape(1, -1)]
    y1, z1, dinv = _run_a(x, latent_vector, style_vector, degp, Ws, F2)
    dinv = dinv.reshape(N)

    Npad = -(-N // 2048) * 2048

    def unpad(t, NC):
        return t.reshape(2, NC, Npad, 64)[:, :, :N]

    NC1 = F2 // 64
    NC2 = O // 64
    t1 = unpad(_sc_edge_scatter(y1.reshape(NC1 * N, 64), src, dst,
                                edge_attr, dinv, N, E, NC1), NC1)
    out1, stats1 = _run_c1(z1, t1, 1000)
    y2, z2 = _run_c2(out1, stats1, W0_g2, W1_g2, b_g2.reshape(1, -1), 1000)

    t2 = unpad(_sc_edge_scatter(y2.reshape(NC2 * N, 64), src, dst,
                                edge_attr, dinv, N, E, NC2), NC2)
    out2, stats2 = _run_c1(z2, t2, 1000)
    h = _run_d2(out2, stats2, 1000)
    return h.reshape(B, R, O)


# speedup vs baseline: 6.4640x; 1.0065x over previous
import functools

import jax
import jax.numpy as jnp
from jax import lax
from jax.experimental import pallas as pl
from jax.experimental.pallas import tpu as pltpu
from jax.experimental.pallas import tpu_sc as plsc

F32 = jnp.float32
_HIGH = lax.Precision.HIGHEST


def _dot(a, b):
    return jnp.dot(a, b, preferred_element_type=F32, precision=_HIGH)


def _lrelu(x):
    return jnp.where(x >= 0, x, 0.2 * x)


def _a_body(x_ref, lat_ref, sty_ref, degp_ref,
            W1_ref, b1_ref, Ws1_ref, bs1_ref,
            W2_ref, b2_ref, Ws2_ref, bs2_ref,
            W3_ref, b3_ref, Ws3_ref, bs3_ref,
            W4_ref, b4_ref, Ws4_ref, bs4_ref,
            W0g_ref, W1g_ref, bg_ref,
            y1_ref, z1_ref, dinv_ref):
    xb = x_ref[...]
    R = xb.shape[0]
    lv = lat_ref[0]
    h = jnp.concatenate([xb, jnp.broadcast_to(lv, (R, lv.shape[1]))], axis=1)
    sv = sty_ref[0]

    def unit(h, W_r, b_r, Ws_r, bs_r):
        t = _dot(h, W_r[...]) + b_r[...]
        mu = jnp.mean(t, axis=0, keepdims=True)
        d = t - mu
        var = jnp.mean(d * d, axis=0, keepdims=True)
        s = _dot(sv, Ws_r[...]) + bs_r[...]
        F = t.shape[1]
        gamma = s[:, :F]
        beta = s[:, F:]
        out = d * lax.rsqrt(var + 1e-5) * (1.0 + gamma) + beta
        return _lrelu(out)

    h = unit(h, W1_ref, b1_ref, Ws1_ref, bs1_ref)
    h = unit(h, W2_ref, b2_ref, Ws2_ref, bs2_ref)
    h = unit(h, W3_ref, b3_ref, Ws3_ref, bs3_ref)
    h = unit(h, W4_ref, b4_ref, Ws4_ref, bs4_ref)

    y1 = _dot(h, W1g_ref[...])
    for cc in range(y1.shape[1] // 64):
        y1_ref[cc] = y1[:, cc * 64:(cc + 1) * 64]
    z1_ref[...] = _dot(h, W0g_ref[...]) + bg_ref[...]

    degb = degp_ref[0, 0] + degp_ref[1, 0]
    dinv_ref[0] = jnp.where(degb > 0, 1.0 / jnp.sqrt(degb + 1e-12), 0.0)


def _run_a(x, lat, sty, degp, Ws, F2):
    N, IN = x.shape
    B, LAT = lat.shape
    R = N // B
    S = sty.shape[1]
    NC = F2 // 64
    full = lambda shape: pl.BlockSpec(shape, lambda b: tuple(0 for _ in shape))
    in_specs = [
        pl.BlockSpec((R, IN), lambda b: (b, 0)),
        pl.BlockSpec((1, 1, LAT), lambda b: (b, 0, 0)),
        pl.BlockSpec((1, 1, S), lambda b: (b, 0, 0)),
        pl.BlockSpec((2, 1, 1, R), lambda b: (0, b, 0, 0)),
    ] + [full(w.shape) for w in Ws]
    out_shapes = [
        jax.ShapeDtypeStruct((NC, N, 64), F32),
        jax.ShapeDtypeStruct((N, F2), F32),
        jax.ShapeDtypeStruct((B, 1, R), F32),
    ]
    out_specs = [
        pl.BlockSpec((NC, R, 64), lambda b: (0, b, 0)),
        pl.BlockSpec((R, F2), lambda b: (b, 0)),
        pl.BlockSpec((1, 1, R), lambda b: (b, 0, 0)),
    ]
    return pl.pallas_call(
        _a_body, grid=(B,), in_specs=in_specs, out_specs=out_specs,
        out_shape=out_shapes,
    )(x, lat.reshape(B, 1, LAT), sty.reshape(B, 1, S),
      degp.reshape(2, B, 1, R), *Ws)


def _lane_bcast(vec, lane):
    idx = jnp.full((16, 1), lane, jnp.int32)
    dnums = lax.GatherDimensionNumbers(
        offset_dims=(), collapsed_slice_dims=(0,), start_index_map=(0,))
    return lax.gather(vec, idx, dnums, (1,),
                      mode=lax.GatherScatterMode.PROMISE_IN_BOUNDS)


def _zero2d(ref, rows, cols):
    z = jnp.zeros((16,), F32)

    def body(r, _):
        for v in range(cols // 16):
            ref[r, pl.ds(v * 16, 16)] = z
        return 0

    lax.fori_loop(0, rows, body, 0)


def _sc_deg(dst, ew, N, E):
    NBLK_SC = E // 256
    per_tile = -(-NBLK_SC // 16)
    mesh = plsc.VectorSubcoreMesh(core_axis_name="c", subcore_axis_name="s")

    @functools.partial(
        pl.kernel, mesh=mesh,
        out_type=jax.ShapeDtypeStruct((2 * N,), F32),
        scratch_types=[
            pltpu.VMEM((1, 128), jnp.int32),
            pltpu.VMEM((128,), F32),
            pltpu.VMEM((N,), F32),
            pltpu.VMEM_SHARED((N,), F32),
        ],
        compiler_params=pltpu.CompilerParams(needs_layout_passes=False),
    )
    def deg_kernel(dst_hbm, ew_hbm, out_hbm, idx_v, val_v, zb_v, acc_sh):
        c = lax.axis_index("c")
        s = lax.axis_index("s")
        z = jnp.zeros((16,), F32)

        def zbody(i, _):
            zb_v[pl.ds(i * 16, 16)] = z
            return 0

        lax.fori_loop(0, N // 16, zbody, 0)

        @pl.when(s == 0)
        def _():
            pltpu.sync_copy(zb_v, acc_sh)

        plsc.subcore_barrier()

        def step(i, _):
            blk = c * NBLK_SC + s + 16 * i

            @pl.when(s + 16 * i < NBLK_SC)
            def _():
                base = blk * 128
                pltpu.sync_copy(dst_hbm.at[pl.ds(base, 128)], idx_v.at[0])
                pltpu.sync_copy(ew_hbm.at[pl.ds(base, 128)], val_v)
                pltpu.sync_copy(val_v, acc_sh.at[idx_v.at[0]], add=True)
            return 0

        lax.fori_loop(0, per_tile, step, 0)
        plsc.subcore_barrier()

        @pl.when(s == 0)
        def _():
            pltpu.sync_copy(acc_sh, zb_v)
            pltpu.sync_copy(zb_v, out_hbm.at[pl.ds(c * N, N)])

    return deg_kernel(dst, ew)


def _sc_edge_scatter(yflat, src, dst, ew, dinv, N, E, NC):
    NBLK_SC = E // 256
    per_tile = -(-NBLK_SC // 16)
    Npad = -(-N // 2048) * 2048
    RPT = Npad // 16
    mesh = plsc.VectorSubcoreMesh(core_axis_name="c", subcore_axis_name="s")

    SB = 2
    SG = -(-per_tile // SB)
    CW = 64

    @functools.partial(
        pl.kernel, mesh=mesh,
        out_type=jax.ShapeDtypeStruct((2 * NC * Npad, CW), F32),
        scratch_types=[
            pltpu.VMEM((N,), F32),
            pltpu.VMEM((SB, 128), jnp.int32),
            pltpu.VMEM((SB, 128), jnp.int32),
            pltpu.VMEM((SB, 128), jnp.int32),
            pltpu.VMEM((SB, 128), F32),
            pltpu.VMEM((SB, 128), F32),
            pltpu.VMEM((SB * 128, CW), F32),
            pltpu.VMEM((128, CW), F32),
            pltpu.VMEM_SHARED((Npad, CW), F32),
            pltpu.SemaphoreType.DMA,
            pltpu.SemaphoreType.DMA,
            pltpu.SemaphoreType.DMA,
        ],
        compiler_params=pltpu.CompilerParams(
            needs_layout_passes=False, use_tc_tiling_on_sc=False),
    )
    def scat_kernel(y_hbm, src_hbm, dst_hbm, ew_hbm, dinv_hbm, out_hbm,
                    dinv_v, src2_v, dst2_v, adj_v, ew_v, norm_v, rbuf,
                    zbuf, acc_sh, esem, gsem, ssem):
        k_core = lax.axis_index("c")
        s = lax.axis_index("s")
        _zero2d(zbuf, 128, CW)
        pltpu.sync_copy(dinv_hbm, dinv_v)
        cnt = lax.div(NBLK_SC - 1 - s, 16) + 1

        def chunk_body(c, _):
            zw = []
            for j in range(RPT // 128):
                zw.append(pltpu.async_copy(
                    zbuf, acc_sh.at[pl.ds(s * RPT + j * 128, 128)], ssem))
            for w in zw:
                w.wait()
            plsc.subcore_barrier()

            def step(g, _):
                stage = []
                bases = []
                for b in range(SB):
                    i_eff = jnp.minimum(g * SB + b, cnt - 1)
                    base = (k_core * NBLK_SC + s + 16 * i_eff) * 128
                    bases.append(base)
                    stage.append(pltpu.async_copy(
                        src_hbm.at[pl.ds(base, 128)], src2_v.at[b], esem))
                    stage.append(pltpu.async_copy(
                        dst_hbm.at[pl.ds(base, 128)], dst2_v.at[b], esem))
                    stage.append(pltpu.async_copy(
                        ew_hbm.at[pl.ds(base, 128)], ew_v.at[b], esem))
                for w in stage:
                    w.wait()
                for b in range(SB):
                    valid = (g * SB + b) < cnt
                    for k in range(8):
                        sl = pl.ds(k * 16, 16)
                        sv = src2_v[b, sl]
                        dv = dst2_v[b, sl]
                        nv = -(ew_v[b, sl]
                               * plsc.load_gather(dinv_v, [sv])
                               * plsc.load_gather(dinv_v, [dv]))
                        norm_v[b, sl] = jnp.where(valid, nv,
                                                  jnp.zeros((16,), F32))
                        adj_v[b, sl] = sv + c * N
                gw = []
                for b in range(SB):
                    gw.append(pltpu.async_copy(
                        y_hbm.at[adj_v.at[b]],
                        rbuf.at[pl.ds(b * 128, 128)], gsem))
                for w in gw:
                    w.wait()
                for b in range(SB):
                    for k in range(8):
                        nv16 = norm_v[b, pl.ds(k * 16, 16)]
                        for lane in range(16):
                            nrm = _lane_bcast(nv16, lane)
                            e = b * 128 + k * 16 + lane
                            for v in range(CW // 16):
                                cs = pl.ds(v * 16, 16)
                                rbuf[e, cs] = rbuf[e, cs] * nrm
                sw = []
                for b in range(SB):
                    sw.append(pltpu.async_copy(
                        rbuf.at[pl.ds(b * 128, 128)],
                        acc_sh.at[dst2_v.at[b]], ssem, add=True))
                for w in sw:
                    w.wait()
                return 0

            lax.fori_loop(0, SG, step, 0)
            plsc.subcore_barrier()
            for j0 in range(0, RPT // 128, SB):
                jn = min(SB, RPT // 128 - j0)
                rw = []
                for j in range(jn):
                    r0 = s * RPT + (j0 + j) * 128
                    rw.append(pltpu.async_copy(
                        acc_sh.at[pl.ds(r0, 128)],
                        rbuf.at[pl.ds(j * 128, 128)], gsem))
                for w in rw:
                    w.wait()
                ww = []
                for j in range(jn):
                    r0 = s * RPT + (j0 + j) * 128
                    ww.append(pltpu.async_copy(
                        rbuf.at[pl.ds(j * 128, 128)],
                        out_hbm.at[pl.ds((k_core * NC + c) * Npad + r0,
                                         128)], gsem))
                for w in ww:
                    w.wait()
            return 0

        lax.fori_loop(0, NC, chunk_body, 0)

    return scat_kernel(yflat, src, dst, ew, dinv)


def _c1_body(z_ref, t_ref, out_ref, stats_ref, acc_ref):
    i = pl.program_id(0)
    nb = pl.num_programs(0)
    t = t_ref[...]
    NC = t.shape[1]
    o = z_ref[...] + jnp.concatenate(
        [t[0, cc] + t[1, cc] for cc in range(NC)], axis=1)
    out_ref[...] = o

    @pl.when(i == 0)
    def _():
        acc_ref[...] = jnp.zeros_like(acc_ref)

    s1 = jnp.sum(o, axis=0, keepdims=True)
    s2 = jnp.sum(o * o, axis=0, keepdims=True)
    acc_ref[...] += jnp.concatenate([s1, s2], axis=0)

    @pl.when(i == nb - 1)
    def _():
        stats_ref[...] = acc_ref[...]


def _run_c1(z, t, RB):
    N, F = z.shape
    NC = t.shape[1]
    CW = t.shape[3]
    grid = (N // RB,)
    return pl.pallas_call(
        _c1_body, grid=grid,
        in_specs=[
            pl.BlockSpec((RB, F), lambda i: (i, 0)),
            pl.BlockSpec((2, NC, RB, CW), lambda i: (0, 0, i, 0)),
        ],
        out_specs=[
            pl.BlockSpec((RB, F), lambda i: (i, 0)),
            pl.BlockSpec((2, F), lambda i: (0, 0)),
        ],
        out_shape=[
            jax.ShapeDtypeStruct((N, F), F32),
            jax.ShapeDtypeStruct((2, F), F32),
        ],
        scratch_shapes=[pltpu.VMEM((2, F), F32)],
    )(z, t)


def _c2_body(o_ref, st_ref, W0_ref, W1_ref, b_ref, y2_ref, z2_ref, *, n):
    st = st_ref[...]
    mu = st[0:1] / n
    var = st[1:2] / n - mu * mu
    h = _lrelu((o_ref[...] - mu) * lax.rsqrt(var + 1e-5))
    y2 = _dot(h, W1_ref[...])
    for cc in range(y2.shape[1] // 64):
        y2_ref[cc] = y2[:, cc * 64:(cc + 1) * 64]
    z2_ref[...] = _dot(h, W0_ref[...]) + b_ref[...]


def _run_c2(out1, stats, W0, W1, b, RB):
    N, F = out1.shape
    O = W0.shape[1]
    NC2 = O // 64
    grid = (N // RB,)
    return pl.pallas_call(
        functools.partial(_c2_body, n=float(N)), grid=grid,
        in_specs=[
            pl.BlockSpec((RB, F), lambda i: (i, 0)),
            pl.BlockSpec((2, F), lambda i: (0, 0)),
            pl.BlockSpec(W0.shape, lambda i: (0, 0)),
            pl.BlockSpec(W1.shape, lambda i: (0, 0)),
            pl.BlockSpec((1, O), lambda i: (0, 0)),
        ],
        out_specs=[
            pl.BlockSpec((NC2, RB, 64), lambda i: (0, i, 0)),
            pl.BlockSpec((RB, O), lambda i: (i, 0)),
        ],
        out_shape=[
            jax.ShapeDtypeStruct((NC2, N, 64), F32),
            jax.ShapeDtypeStruct((N, O), F32),
        ],
    )(out1, stats, W0, W1, b)


def _d2_body(o_ref, st_ref, out_ref, *, n):
    st = st_ref[...]
    mu = st[0:1] / n
    var = st[1:2] / n - mu * mu
    out_ref[...] = _lrelu((o_ref[...] - mu) * lax.rsqrt(var + 1e-5))


def _run_d2(out2, stats, RB):
    N, F = out2.shape
    grid = (N // RB,)
    return pl.pallas_call(
        functools.partial(_d2_body, n=float(N)), grid=grid,
        in_specs=[
            pl.BlockSpec((RB, F), lambda i: (i, 0)),
            pl.BlockSpec((2, F), lambda i: (0, 0)),
        ],
        out_specs=pl.BlockSpec((RB, F), lambda i: (i, 0)),
        out_shape=jax.ShapeDtypeStruct((N, F), F32),
    )(out2, stats)


def kernel(x, latent_vector, style_vector, edge_index, edge_attr, batch_size,
           nroi, W_fc1, b_fc1, Ws_fc1, bs_fc1, W_fc2, b_fc2, Ws_fc2, bs_fc2,
           W_fc3, b_fc3, Ws_fc3, bs_fc3, W_fc4, b_fc4, Ws_fc4, bs_fc4,
           W0_g1, W1_g1, b_g1, W0_g2, W1_g2, b_g2):
    N, IN = x.shape
    B, LAT = latent_vector.shape
    R = N // B
    E = edge_index.shape[1]
    F2 = W_fc4.shape[1]
    O = W0_g2.shape[1]
    src = edge_index[0]
    dst = edge_index[1]

    degp = _sc_deg(dst, edge_attr, N, E).reshape(2, B, R)

    Ws = [W_fc1, b_fc1.reshape(1, -1), Ws_fc1, bs_fc1.reshape(1, -1),
          W_fc2, b_fc2.reshape(1, -1), Ws_fc2, bs_fc2.reshape(1, -1),
          W_fc3, b_fc3.reshape(1, -1), Ws_fc3, bs_fc3.reshape(1, -1),
          W_fc4, b_fc4.reshape(1, -1), Ws_fc4, bs_fc4.reshape(1, -1),
          W0_g1, W1_g1, b_g1.reshape(1, -1)]
    y1, z1, dinv = _run_a(x, latent_vector, style_vector, degp, Ws, F2)
    dinv = dinv.reshape(N)

    Npad = -(-N // 2048) * 2048

    def unpad(t, NC):
        return t.reshape(2, NC, Npad, 64)[:, :, :N]

    NC1 = F2 // 64
    NC2 = O // 64
    t1 = unpad(_sc_edge_scatter(y1.reshape(NC1 * N, 64), src, dst,
                                edge_attr, dinv, N, E, NC1), NC1)
    out1, stats1 = _run_c1(z1, t1, 1000)
    y2, z2 = _run_c2(out1, stats1, W0_g2, W1_g2, b_g2.reshape(1, -1), 1000)

    t2 = unpad(_sc_edge_scatter(y2.reshape(NC2 * N, 64), src, dst,
                                edge_attr, dinv, N, E, NC2), NC2)
    out2, stats2 = _run_c1(z2, t2, 1000)
    h = _run_d2(out2, stats2, 1000)
    return h.reshape(B, R, O)


# speedup vs baseline: 7.8982x; 1.2219x over previous
import functools

import jax
import jax.numpy as jnp
from jax import lax
from jax.experimental import pallas as pl
from jax.experimental.pallas import tpu as pltpu
from jax.experimental.pallas import tpu_sc as plsc

F32 = jnp.float32
_HIGH = lax.Precision.HIGHEST


def _dot(a, b):
    return jnp.dot(a, b, preferred_element_type=F32, precision=_HIGH)


def _lrelu(x):
    return jnp.where(x >= 0, x, 0.2 * x)


def _a_body(x_ref, lat_ref, sty_ref, degp_ref,
            W1_ref, b1_ref, Ws1_ref, bs1_ref,
            W2_ref, b2_ref, Ws2_ref, bs2_ref,
            W3_ref, b3_ref, Ws3_ref, bs3_ref,
            W4_ref, b4_ref, Ws4_ref, bs4_ref,
            W0g_ref, W1g_ref, bg_ref,
            y1_ref, z1_ref, dinv_ref):
    xb = x_ref[...]
    R = xb.shape[0]
    lv = lat_ref[0]
    h = jnp.concatenate([xb, jnp.broadcast_to(lv, (R, lv.shape[1]))], axis=1)
    sv = sty_ref[0]

    def unit(h, W_r, b_r, Ws_r, bs_r):
        t = _dot(h, W_r[...]) + b_r[...]
        mu = jnp.mean(t, axis=0, keepdims=True)
        d = t - mu
        var = jnp.mean(d * d, axis=0, keepdims=True)
        s = _dot(sv, Ws_r[...]) + bs_r[...]
        F = t.shape[1]
        gamma = s[:, :F]
        beta = s[:, F:]
        out = d * lax.rsqrt(var + 1e-5) * (1.0 + gamma) + beta
        return _lrelu(out)

    h = unit(h, W1_ref, b1_ref, Ws1_ref, bs1_ref)
    h = unit(h, W2_ref, b2_ref, Ws2_ref, bs2_ref)
    h = unit(h, W3_ref, b3_ref, Ws3_ref, bs3_ref)
    h = unit(h, W4_ref, b4_ref, Ws4_ref, bs4_ref)

    y1 = _dot(h, W1g_ref[...])
    for cc in range(y1.shape[1] // 64):
        y1_ref[cc] = y1[:, cc * 64:(cc + 1) * 64]
    z1_ref[...] = _dot(h, W0g_ref[...]) + bg_ref[...]

    degb = degp_ref[0, 0] + degp_ref[1, 0]
    dinv_ref[0] = jnp.where(degb > 0, 1.0 / jnp.sqrt(degb + 1e-12), 0.0)


def _run_a(x, lat, sty, degp, Ws, F2):
    N, IN = x.shape
    B, LAT = lat.shape
    R = N // B
    S = sty.shape[1]
    NC = F2 // 64
    full = lambda shape: pl.BlockSpec(shape, lambda b: tuple(0 for _ in shape))
    in_specs = [
        pl.BlockSpec((R, IN), lambda b: (b, 0)),
        pl.BlockSpec((1, 1, LAT), lambda b: (b, 0, 0)),
        pl.BlockSpec((1, 1, S), lambda b: (b, 0, 0)),
        pl.BlockSpec((2, 1, 1, R), lambda b: (0, b, 0, 0)),
    ] + [full(w.shape) for w in Ws]
    out_shapes = [
        jax.ShapeDtypeStruct((NC, N, 64), F32),
        jax.ShapeDtypeStruct((N, F2), F32),
        jax.ShapeDtypeStruct((B, 1, R), F32),
    ]
    out_specs = [
        pl.BlockSpec((NC, R, 64), lambda b: (0, b, 0)),
        pl.BlockSpec((R, F2), lambda b: (b, 0)),
        pl.BlockSpec((1, 1, R), lambda b: (b, 0, 0)),
    ]
    return pl.pallas_call(
        _a_body, grid=(B,), in_specs=in_specs, out_specs=out_specs,
        out_shape=out_shapes,
    )(x, lat.reshape(B, 1, LAT), sty.reshape(B, 1, S),
      degp.reshape(2, B, 1, R), *Ws)


def _lane_bcast(vec, lane):
    idx = jnp.full((16, 1), lane, jnp.int32)
    dnums = lax.GatherDimensionNumbers(
        offset_dims=(), collapsed_slice_dims=(0,), start_index_map=(0,))
    return lax.gather(vec, idx, dnums, (1,),
                      mode=lax.GatherScatterMode.PROMISE_IN_BOUNDS)


def _zero2d(ref, rows, cols):
    z = jnp.zeros((16,), F32)

    def body(r, _):
        for v in range(cols // 16):
            ref[r, pl.ds(v * 16, 16)] = z
        return 0

    lax.fori_loop(0, rows, body, 0)


def _sc_deg(dst, ew, N, E):
    NBLK_SC = E // 256
    per_tile = -(-NBLK_SC // 16)
    mesh = plsc.VectorSubcoreMesh(core_axis_name="c", subcore_axis_name="s")

    @functools.partial(
        pl.kernel, mesh=mesh,
        out_type=jax.ShapeDtypeStruct((2 * N,), F32),
        scratch_types=[
            pltpu.VMEM((1, 128), jnp.int32),
            pltpu.VMEM((128,), F32),
            pltpu.VMEM((N,), F32),
            pltpu.VMEM_SHARED((N,), F32),
        ],
        compiler_params=pltpu.CompilerParams(needs_layout_passes=False),
    )
    def deg_kernel(dst_hbm, ew_hbm, out_hbm, idx_v, val_v, zb_v, acc_sh):
        c = lax.axis_index("c")
        s = lax.axis_index("s")
        z = jnp.zeros((16,), F32)

        def zbody(i, _):
            zb_v[pl.ds(i * 16, 16)] = z
            return 0

        lax.fori_loop(0, N // 16, zbody, 0)

        @pl.when(s == 0)
        def _():
            pltpu.sync_copy(zb_v, acc_sh)

        plsc.subcore_barrier()

        def step(i, _):
            blk = c * NBLK_SC + s + 16 * i

            @pl.when(s + 16 * i < NBLK_SC)
            def _():
                base = blk * 128
                pltpu.sync_copy(dst_hbm.at[pl.ds(base, 128)], idx_v.at[0])
                pltpu.sync_copy(ew_hbm.at[pl.ds(base, 128)], val_v)
                pltpu.sync_copy(val_v, acc_sh.at[idx_v.at[0]], add=True)
            return 0

        lax.fori_loop(0, per_tile, step, 0)
        plsc.subcore_barrier()

        @pl.when(s == 0)
        def _():
            pltpu.sync_copy(acc_sh, zb_v)
            pltpu.sync_copy(zb_v, out_hbm.at[pl.ds(c * N, N)])

    return deg_kernel(dst, ew)


def _sc_edge_scatter(yflat, src, dst, ew, dinv, N, E, NC):
    NBLK_SC = E // 256
    per_tile = -(-NBLK_SC // 16)
    Npad = -(-N // 2048) * 2048
    RPT = Npad // 16
    mesh = plsc.VectorSubcoreMesh(core_axis_name="c", subcore_axis_name="s")

    SB = 2
    SG = -(-per_tile // SB)
    SG2 = -(-SG // 2)
    CW = 64
    NBUF = 2 * SB

    @functools.partial(
        pl.kernel, mesh=mesh,
        out_type=jax.ShapeDtypeStruct((2 * NC * Npad, CW), F32),
        scratch_types=[
            pltpu.VMEM((N,), F32),
            pltpu.VMEM((NBUF, 128), jnp.int32),
            pltpu.VMEM((NBUF, 128), jnp.int32),
            pltpu.VMEM((NBUF, 128), jnp.int32),
            pltpu.VMEM((NBUF, 128), jnp.int32),
            pltpu.VMEM((NBUF, 128), F32),
            pltpu.VMEM((NBUF, 128), F32),
            pltpu.VMEM((NBUF * 128, CW), F32),
            pltpu.VMEM((128, CW), F32),
            pltpu.VMEM_SHARED((Npad, CW), F32),
            pltpu.SemaphoreType.DMA,
            pltpu.SemaphoreType.DMA,
            pltpu.SemaphoreType.DMA,
        ],
        compiler_params=pltpu.CompilerParams(
            needs_layout_passes=False, use_tc_tiling_on_sc=False),
    )
    def scat_kernel(y_hbm, src_hbm, dst_hbm, ew_hbm, dinv_hbm, out_hbm,
                    dinv_v, src2_v, dst2_v, sdst_v, adj_v, ew_v, norm_v,
                    rbuf, zbuf, acc_sh, esem, gsem, ssem):
        k_core = lax.axis_index("c")
        s = lax.axis_index("s")
        _zero2d(zbuf, 128, CW)
        pltpu.sync_copy(dinv_hbm, dinv_v)
        cnt = lax.div(NBLK_SC - 1 - s, 16) + 1

        def fire_stage(g, p):
            for b in range(SB):
                i_eff = jnp.minimum(g * SB + b, cnt - 1)
                base = (k_core * NBLK_SC + s + 16 * i_eff) * 128
                r = p * SB + b
                pltpu.async_copy(src_hbm.at[pl.ds(base, 128)],
                                 src2_v.at[r], esem)
                pltpu.async_copy(dst_hbm.at[pl.ds(base, 128)],
                                 dst2_v.at[r], esem)
                pltpu.async_copy(ew_hbm.at[pl.ds(base, 128)],
                                 ew_v.at[r], esem)

        def drain_stage(p):
            for b in range(SB):
                r = p * SB + b
                pltpu.make_async_copy(src_hbm.at[pl.ds(0, 128)],
                                      src2_v.at[r], esem).wait()
                pltpu.make_async_copy(dst_hbm.at[pl.ds(0, 128)],
                                      dst2_v.at[r], esem).wait()
                pltpu.make_async_copy(ew_hbm.at[pl.ds(0, 128)],
                                      ew_v.at[r], esem).wait()

        def norms(g, p, c):
            for b in range(SB):
                r = p * SB + b
                valid = (g * SB + b) < cnt
                for k in range(8):
                    sl = pl.ds(k * 16, 16)
                    sv = src2_v[r, sl]
                    dv = dst2_v[r, sl]
                    nv = -(ew_v[r, sl]
                           * plsc.load_gather(dinv_v, [sv])
                           * plsc.load_gather(dinv_v, [dv]))
                    norm_v[r, sl] = jnp.where(valid, nv,
                                              jnp.zeros((16,), F32))
                    adj_v[r, sl] = sv + c * N

        def fire_gather(p):
            for b in range(SB):
                r = p * SB + b
                pltpu.async_copy(y_hbm.at[adj_v.at[r]],
                                 rbuf.at[pl.ds(r * 128, 128)], gsem)

        def drain_gather(p):
            for b in range(SB):
                r = p * SB + b
                pltpu.make_async_copy(y_hbm.at[adj_v.at[r]],
                                      rbuf.at[pl.ds(r * 128, 128)],
                                      gsem).wait()

        def scale(p):
            for b in range(SB):
                r = p * SB + b
                for k in range(8):
                    nv16 = norm_v[r, pl.ds(k * 16, 16)]

                    def body(lane, _):
                        nrm = _lane_bcast(nv16, lane)
                        e = r * 128 + k * 16 + lane
                        for v in range(CW // 16):
                            cs = pl.ds(v * 16, 16)
                            rbuf[e, cs] = rbuf[e, cs] * nrm
                        return 0

                    lax.fori_loop(0, 16, body, 0)

        def snap_and_fire_scatter(p):
            for b in range(SB):
                r = p * SB + b
                for k in range(8):
                    sl = pl.ds(k * 16, 16)
                    sdst_v[r, sl] = dst2_v[r, sl]
            for b in range(SB):
                r = p * SB + b
                pltpu.async_copy(rbuf.at[pl.ds(r * 128, 128)],
                                 acc_sh.at[sdst_v.at[r]], ssem, add=True)

        def drain_scatter(p):
            for b in range(SB):
                r = p * SB + b
                pltpu.make_async_copy(rbuf.at[pl.ds(r * 128, 128)],
                                      acc_sh.at[sdst_v.at[r]], ssem).wait()

        def chunk_body(c, _):
            zw = []
            for j in range(RPT // 128):
                zw.append(pltpu.async_copy(
                    zbuf, acc_sh.at[pl.ds(s * RPT + j * 128, 128)], ssem))
            for w in zw:
                w.wait()
            plsc.subcore_barrier()

            fire_stage(0, 0)
            drain_stage(0)
            norms(0, 0, c)
            fire_gather(0)
            fire_stage(1, 1)

            def step(t, _):
                g0 = 2 * t
                g1 = 2 * t + 1
                drain_stage(1)
                norms(g1, 1, c)
                drain_gather(0)
                fire_gather(1)
                scale(0)
                snap_and_fire_scatter(0)
                fire_stage(g0 + 2, 0)
                drain_stage(0)
                norms(g0 + 2, 0, c)
                drain_gather(1)
                scale(1)
                snap_and_fire_scatter(1)
                drain_scatter(0)
                fire_gather(0)
                fire_stage(g1 + 2, 1)
                drain_scatter(1)
                return 0

            lax.fori_loop(0, SG2, step, 0)
            drain_gather(0)
            drain_stage(1)
            plsc.subcore_barrier()
            for j0 in range(0, RPT // 128, 4):
                jn = min(4, RPT // 128 - j0)
                rw = []
                for j in range(jn):
                    r0 = s * RPT + (j0 + j) * 128
                    rw.append(pltpu.async_copy(
                        acc_sh.at[pl.ds(r0, 128)],
                        rbuf.at[pl.ds(j * 128, 128)], gsem))
                for w in rw:
                    w.wait()
                ww = []
                for j in range(jn):
                    r0 = s * RPT + (j0 + j) * 128
                    ww.append(pltpu.async_copy(
                        rbuf.at[pl.ds(j * 128, 128)],
                        out_hbm.at[pl.ds((k_core * NC + c) * Npad + r0,
                                         128)], gsem))
                for w in ww:
                    w.wait()
            return 0

        lax.fori_loop(0, NC, chunk_body, 0)

    return scat_kernel(yflat, src, dst, ew, dinv)


def _c1_body(z_ref, t_ref, out_ref, stats_ref, acc_ref):
    i = pl.program_id(0)
    nb = pl.num_programs(0)
    t = t_ref[...]
    NC = t.shape[1]
    o = z_ref[...] + jnp.concatenate(
        [t[0, cc] + t[1, cc] for cc in range(NC)], axis=1)
    out_ref[...] = o

    @pl.when(i == 0)
    def _():
        acc_ref[...] = jnp.zeros_like(acc_ref)

    s1 = jnp.sum(o, axis=0, keepdims=True)
    s2 = jnp.sum(o * o, axis=0, keepdims=True)
    acc_ref[...] += jnp.concatenate([s1, s2], axis=0)

    @pl.when(i == nb - 1)
    def _():
        stats_ref[...] = acc_ref[...]


def _run_c1(z, t, RB):
    N, F = z.shape
    NC = t.shape[1]
    CW = t.shape[3]
    grid = (N // RB,)
    return pl.pallas_call(
        _c1_body, grid=grid,
        in_specs=[
            pl.BlockSpec((RB, F), lambda i: (i, 0)),
            pl.BlockSpec((2, NC, RB, CW), lambda i: (0, 0, i, 0)),
        ],
        out_specs=[
            pl.BlockSpec((RB, F), lambda i: (i, 0)),
            pl.BlockSpec((2, F), lambda i: (0, 0)),
        ],
        out_shape=[
            jax.ShapeDtypeStruct((N, F), F32),
            jax.ShapeDtypeStruct((2, F), F32),
        ],
        scratch_shapes=[pltpu.VMEM((2, F), F32)],
    )(z, t)


def _c2_body(o_ref, st_ref, W0_ref, W1_ref, b_ref, y2_ref, z2_ref, *, n):
    st = st_ref[...]
    mu = st[0:1] / n
    var = st[1:2] / n - mu * mu
    h = _lrelu((o_ref[...] - mu) * lax.rsqrt(var + 1e-5))
    y2 = _dot(h, W1_ref[...])
    for cc in range(y2.shape[1] // 64):
        y2_ref[cc] = y2[:, cc * 64:(cc + 1) * 64]
    z2_ref[...] = _dot(h, W0_ref[...]) + b_ref[...]


def _run_c2(out1, stats, W0, W1, b, RB):
    N, F = out1.shape
    O = W0.shape[1]
    NC2 = O // 64
    grid = (N // RB,)
    return pl.pallas_call(
        functools.partial(_c2_body, n=float(N)), grid=grid,
        in_specs=[
            pl.BlockSpec((RB, F), lambda i: (i, 0)),
            pl.BlockSpec((2, F), lambda i: (0, 0)),
            pl.BlockSpec(W0.shape, lambda i: (0, 0)),
            pl.BlockSpec(W1.shape, lambda i: (0, 0)),
            pl.BlockSpec((1, O), lambda i: (0, 0)),
        ],
        out_specs=[
            pl.BlockSpec((NC2, RB, 64), lambda i: (0, i, 0)),
            pl.BlockSpec((RB, O), lambda i: (i, 0)),
        ],
        out_shape=[
            jax.ShapeDtypeStruct((NC2, N, 64), F32),
            jax.ShapeDtypeStruct((N, O), F32),
        ],
    )(out1, stats, W0, W1, b)


def _d2_body(o_ref, st_ref, out_ref, *, n):
    st = st_ref[...]
    mu = st[0:1] / n
    var = st[1:2] / n - mu * mu
    out_ref[...] = _lrelu((o_ref[...] - mu) * lax.rsqrt(var + 1e-5))


def _run_d2(out2, stats, RB):
    N, F = out2.shape
    grid = (N // RB,)
    return pl.pallas_call(
        functools.partial(_d2_body, n=float(N)), grid=grid,
        in_specs=[
            pl.BlockSpec((RB, F), lambda i: (i, 0)),
            pl.BlockSpec((2, F), lambda i: (0, 0)),
        ],
        out_specs=pl.BlockSpec((RB, F), lambda i: (i, 0)),
        out_shape=jax.ShapeDtypeStruct((N, F), F32),
    )(out2, stats)


def kernel(x, latent_vector, style_vector, edge_index, edge_attr, batch_size,
           nroi, W_fc1, b_fc1, Ws_fc1, bs_fc1, W_fc2, b_fc2, Ws_fc2, bs_fc2,
           W_fc3, b_fc3, Ws_fc3, bs_fc3, W_fc4, b_fc4, Ws_fc4, bs_fc4,
           W0_g1, W1_g1, b_g1, W0_g2, W1_g2, b_g2):
    N, IN = x.shape
    B, LAT = latent_vector.shape
    R = N // B
    E = edge_index.shape[1]
    F2 = W_fc4.shape[1]
    O = W0_g2.shape[1]
    src = edge_index[0]
    dst = edge_index[1]

    degp = _sc_deg(dst, edge_attr, N, E).reshape(2, B, R)

    Ws = [W_fc1, b_fc1.reshape(1, -1), Ws_fc1, bs_fc1.reshape(1, -1),
          W_fc2, b_fc2.reshape(1, -1), Ws_fc2, bs_fc2.reshape(1, -1),
          W_fc3, b_fc3.reshape(1, -1), Ws_fc3, bs_fc3.reshape(1, -1),
          W_fc4, b_fc4.reshape(1, -1), Ws_fc4, bs_fc4.reshape(1, -1),
          W0_g1, W1_g1, b_g1.reshape(1, -1)]
    y1, z1, dinv = _run_a(x, latent_vector, style_vector, degp, Ws, F2)
    dinv = dinv.reshape(N)

    Npad = -(-N // 2048) * 2048

    def unpad(t, NC):
        return t.reshape(2, NC, Npad, 64)[:, :, :N]

    NC1 = F2 // 64
    NC2 = O // 64
    t1 = unpad(_sc_edge_scatter(y1.reshape(NC1 * N, 64), src, dst,
                                edge_attr, dinv, N, E, NC1), NC1)
    out1, stats1 = _run_c1(z1, t1, 1000)
    y2, z2 = _run_c2(out1, stats1, W0_g2, W1_g2, b_g2.reshape(1, -1), 1000)

    t2 = unpad(_sc_edge_scatter(y2.reshape(NC2 * N, 64), src, dst,
                                edge_attr, dinv, N, E, NC2), NC2)
    out2, stats2 = _run_c1(z2, t2, 1000)
    h = _run_d2(out2, stats2, 1000)
    return h.reshape(B, R, O)


# speedup vs baseline: 8.9224x; 1.1297x over previous
import functools

import jax
import jax.numpy as jnp
from jax import lax
from jax.experimental import pallas as pl
from jax.experimental.pallas import tpu as pltpu
from jax.experimental.pallas import tpu_sc as plsc

F32 = jnp.float32
_HIGH = lax.Precision.HIGHEST


def _dot(a, b):
    return jnp.dot(a, b, preferred_element_type=F32)


def _lrelu(x):
    return jnp.where(x >= 0, x, 0.2 * x)


def _a_body(x_ref, lat_ref, sty_ref, degp_ref,
            W1_ref, b1_ref, Ws1_ref, bs1_ref,
            W2_ref, b2_ref, Ws2_ref, bs2_ref,
            W3_ref, b3_ref, Ws3_ref, bs3_ref,
            W4_ref, b4_ref, Ws4_ref, bs4_ref,
            W0g_ref, W1g_ref, bg_ref,
            y1_ref, z1_ref, dinv_ref):
    xb = x_ref[...]
    R = xb.shape[0]
    lv = lat_ref[0]
    h = jnp.concatenate([xb, jnp.broadcast_to(lv, (R, lv.shape[1]))], axis=1)
    sv = sty_ref[0]

    def unit(h, W_r, b_r, Ws_r, bs_r):
        t = _dot(h, W_r[...]) + b_r[...]
        mu = jnp.mean(t, axis=0, keepdims=True)
        d = t - mu
        var = jnp.mean(d * d, axis=0, keepdims=True)
        s = _dot(sv, Ws_r[...]) + bs_r[...]
        F = t.shape[1]
        gamma = s[:, :F]
        beta = s[:, F:]
        out = d * lax.rsqrt(var + 1e-5) * (1.0 + gamma) + beta
        return _lrelu(out)

    h = unit(h, W1_ref, b1_ref, Ws1_ref, bs1_ref)
    h = unit(h, W2_ref, b2_ref, Ws2_ref, bs2_ref)
    h = unit(h, W3_ref, b3_ref, Ws3_ref, bs3_ref)
    h = unit(h, W4_ref, b4_ref, Ws4_ref, bs4_ref)

    y1 = _dot(h, W1g_ref[...])
    for cc in range(y1.shape[1] // 64):
        y1_ref[cc] = y1[:, cc * 64:(cc + 1) * 64]
    z1_ref[...] = _dot(h, W0g_ref[...]) + bg_ref[...]

    degb = degp_ref[0, 0] + degp_ref[1, 0]
    dinv_ref[0] = jnp.where(degb > 0, 1.0 / jnp.sqrt(degb + 1e-12), 0.0)


def _run_a(x, lat, sty, degp, Ws, F2):
    N, IN = x.shape
    B, LAT = lat.shape
    R = N // B
    S = sty.shape[1]
    NC = F2 // 64
    full = lambda shape: pl.BlockSpec(shape, lambda b: tuple(0 for _ in shape))
    in_specs = [
        pl.BlockSpec((R, IN), lambda b: (b, 0)),
        pl.BlockSpec((1, 1, LAT), lambda b: (b, 0, 0)),
        pl.BlockSpec((1, 1, S), lambda b: (b, 0, 0)),
        pl.BlockSpec((2, 1, 1, R), lambda b: (0, b, 0, 0)),
    ] + [full(w.shape) for w in Ws]
    out_shapes = [
        jax.ShapeDtypeStruct((NC, N, 64), F32),
        jax.ShapeDtypeStruct((N, F2), F32),
        jax.ShapeDtypeStruct((B, 1, R), F32),
    ]
    out_specs = [
        pl.BlockSpec((NC, R, 64), lambda b: (0, b, 0)),
        pl.BlockSpec((R, F2), lambda b: (b, 0)),
        pl.BlockSpec((1, 1, R), lambda b: (b, 0, 0)),
    ]
    return pl.pallas_call(
        _a_body, grid=(B,), in_specs=in_specs, out_specs=out_specs,
        out_shape=out_shapes,
    )(x, lat.reshape(B, 1, LAT), sty.reshape(B, 1, S),
      degp.reshape(2, B, 1, R), *Ws)


def _lane_bcast(vec, lane):
    idx = jnp.full((16, 1), lane, jnp.int32)
    dnums = lax.GatherDimensionNumbers(
        offset_dims=(), collapsed_slice_dims=(0,), start_index_map=(0,))
    return lax.gather(vec, idx, dnums, (1,),
                      mode=lax.GatherScatterMode.PROMISE_IN_BOUNDS)


def _zero2d(ref, rows, cols):
    z = jnp.zeros((16,), F32)

    def body(r, _):
        for v in range(cols // 16):
            ref[r, pl.ds(v * 16, 16)] = z
        return 0

    lax.fori_loop(0, rows, body, 0)


def _sc_deg(dst, ew, N, E):
    NBLK_SC = E // 256
    per_tile = -(-NBLK_SC // 16)
    mesh = plsc.VectorSubcoreMesh(core_axis_name="c", subcore_axis_name="s")

    @functools.partial(
        pl.kernel, mesh=mesh,
        out_type=jax.ShapeDtypeStruct((2 * N,), F32),
        scratch_types=[
            pltpu.VMEM((1, 128), jnp.int32),
            pltpu.VMEM((128,), F32),
            pltpu.VMEM((N,), F32),
            pltpu.VMEM_SHARED((N,), F32),
        ],
        compiler_params=pltpu.CompilerParams(needs_layout_passes=False),
    )
    def deg_kernel(dst_hbm, ew_hbm, out_hbm, idx_v, val_v, zb_v, acc_sh):
        c = lax.axis_index("c")
        s = lax.axis_index("s")
        z = jnp.zeros((16,), F32)

        def zbody(i, _):
            zb_v[pl.ds(i * 16, 16)] = z
            return 0

        lax.fori_loop(0, N // 16, zbody, 0)

        @pl.when(s == 0)
        def _():
            pltpu.sync_copy(zb_v, acc_sh)

        plsc.subcore_barrier()

        def step(i, _):
            blk = c * NBLK_SC + s + 16 * i

            @pl.when(s + 16 * i < NBLK_SC)
            def _():
                base = blk * 128
                pltpu.sync_copy(dst_hbm.at[pl.ds(base, 128)], idx_v.at[0])
                pltpu.sync_copy(ew_hbm.at[pl.ds(base, 128)], val_v)
                pltpu.sync_copy(val_v, acc_sh.at[idx_v.at[0]], add=True)
            return 0

        lax.fori_loop(0, per_tile, step, 0)
        plsc.subcore_barrier()

        @pl.when(s == 0)
        def _():
            pltpu.sync_copy(acc_sh, zb_v)
            pltpu.sync_copy(zb_v, out_hbm.at[pl.ds(c * N, N)])

    return deg_kernel(dst, ew)


def _sc_edge_scatter(yflat, src, dst, ew, dinv, N, E, NC):
    NBLK_SC = E // 256
    per_tile = -(-NBLK_SC // 16)
    Npad = -(-N // 2048) * 2048
    RPT = Npad // 16
    mesh = plsc.VectorSubcoreMesh(core_axis_name="c", subcore_axis_name="s")

    SB = 2
    SG = -(-per_tile // SB)
    SG2 = -(-SG // 2)
    CW = 64
    NBUF = 2 * SB

    @functools.partial(
        pl.kernel, mesh=mesh,
        out_type=jax.ShapeDtypeStruct((2 * NC * Npad, CW), F32),
        scratch_types=[
            pltpu.VMEM((N,), F32),
            pltpu.VMEM((NBUF, 128), jnp.int32),
            pltpu.VMEM((NBUF, 128), jnp.int32),
            pltpu.VMEM((NBUF, 128), jnp.int32),
            pltpu.VMEM((NBUF, 128), jnp.int32),
            pltpu.VMEM((NBUF, 128), F32),
            pltpu.VMEM((NBUF, 128), F32),
            pltpu.VMEM((NBUF * 128, CW), F32),
            pltpu.VMEM((128, CW), F32),
            pltpu.VMEM_SHARED((Npad, CW), F32),
            pltpu.SemaphoreType.DMA,
            pltpu.SemaphoreType.DMA,
            pltpu.SemaphoreType.DMA,
        ],
        compiler_params=pltpu.CompilerParams(
            needs_layout_passes=False, use_tc_tiling_on_sc=False),
    )
    def scat_kernel(y_hbm, src_hbm, dst_hbm, ew_hbm, dinv_hbm, out_hbm,
                    dinv_v, src2_v, dst2_v, sdst_v, adj_v, ew_v, norm_v,
                    rbuf, zbuf, acc_sh, esem, gsem, ssem):
        k_core = lax.axis_index("c")
        s = lax.axis_index("s")
        _zero2d(zbuf, 128, CW)
        pltpu.sync_copy(dinv_hbm, dinv_v)
        cnt = lax.div(NBLK_SC - 1 - s, 16) + 1

        def fire_stage(g, p):
            for b in range(SB):
                i_eff = jnp.minimum(g * SB + b, cnt - 1)
                base = (k_core * NBLK_SC + s + 16 * i_eff) * 128
                r = p * SB + b
                pltpu.async_copy(src_hbm.at[pl.ds(base, 128)],
                                 src2_v.at[r], esem)
                pltpu.async_copy(dst_hbm.at[pl.ds(base, 128)],
                                 dst2_v.at[r], esem)
                pltpu.async_copy(ew_hbm.at[pl.ds(base, 128)],
                                 ew_v.at[r], esem)

        def drain_stage(p):
            for b in range(SB):
                r = p * SB + b
                pltpu.make_async_copy(src_hbm.at[pl.ds(0, 128)],
                                      src2_v.at[r], esem).wait()
                pltpu.make_async_copy(dst_hbm.at[pl.ds(0, 128)],
                                      dst2_v.at[r], esem).wait()
                pltpu.make_async_copy(ew_hbm.at[pl.ds(0, 128)],
                                      ew_v.at[r], esem).wait()

        def norms(g, p, c):
            for b in range(SB):
                r = p * SB + b
                valid = (g * SB + b) < cnt
                for k in range(8):
                    sl = pl.ds(k * 16, 16)
                    sv = src2_v[r, sl]
                    dv = dst2_v[r, sl]
                    nv = -(ew_v[r, sl]
                           * plsc.load_gather(dinv_v, [sv])
                           * plsc.load_gather(dinv_v, [dv]))
                    norm_v[r, sl] = jnp.where(valid, nv,
                                              jnp.zeros((16,), F32))
                    adj_v[r, sl] = sv + c * N

        def fire_gather(p):
            for b in range(SB):
                r = p * SB + b
                pltpu.async_copy(y_hbm.at[adj_v.at[r]],
                                 rbuf.at[pl.ds(r * 128, 128)], gsem)

        def drain_gather(p):
            for b in range(SB):
                r = p * SB + b
                pltpu.make_async_copy(y_hbm.at[adj_v.at[r]],
                                      rbuf.at[pl.ds(r * 128, 128)],
                                      gsem).wait()

        def scale(p):
            for b in range(SB):
                r = p * SB + b
                for k in range(8):
                    nv16 = norm_v[r, pl.ds(k * 16, 16)]

                    def body(lane, _):
                        nrm = _lane_bcast(nv16, lane)
                        e = r * 128 + k * 16 + lane
                        for v in range(CW // 16):
                            cs = pl.ds(v * 16, 16)
                            rbuf[e, cs] = rbuf[e, cs] * nrm
                        return 0

                    lax.fori_loop(0, 16, body, 0)

        def snap_and_fire_scatter(p):
            for b in range(SB):
                r = p * SB + b
                for k in range(8):
                    sl = pl.ds(k * 16, 16)
                    sdst_v[r, sl] = dst2_v[r, sl]
            for b in range(SB):
                r = p * SB + b
                pltpu.async_copy(rbuf.at[pl.ds(r * 128, 128)],
                                 acc_sh.at[sdst_v.at[r]], ssem, add=True)

        def drain_scatter(p):
            for b in range(SB):
                r = p * SB + b
                pltpu.make_async_copy(rbuf.at[pl.ds(r * 128, 128)],
                                      acc_sh.at[sdst_v.at[r]], ssem).wait()

        def chunk_body(c, _):
            zw = []
            for j in range(RPT // 128):
                zw.append(pltpu.async_copy(
                    zbuf, acc_sh.at[pl.ds(s * RPT + j * 128, 128)], ssem))
            for w in zw:
                w.wait()
            plsc.subcore_barrier()

            fire_stage(0, 0)
            drain_stage(0)
            norms(0, 0, c)
            fire_gather(0)
            fire_stage(1, 1)

            def step(t, _):
                g0 = 2 * t
                g1 = 2 * t + 1
                drain_stage(1)
                norms(g1, 1, c)
                drain_gather(0)
                fire_gather(1)
                scale(0)
                snap_and_fire_scatter(0)
                fire_stage(g0 + 2, 0)
                drain_stage(0)
                norms(g0 + 2, 0, c)
                drain_gather(1)
                scale(1)
                snap_and_fire_scatter(1)
                drain_scatter(0)
                fire_gather(0)
                fire_stage(g1 + 2, 1)
                drain_scatter(1)
                return 0

            lax.fori_loop(0, SG2, step, 0)
            drain_gather(0)
            drain_stage(1)
            plsc.subcore_barrier()
            for j0 in range(0, RPT // 128, 4):
                jn = min(4, RPT // 128 - j0)
                rw = []
                for j in range(jn):
                    r0 = s * RPT + (j0 + j) * 128
                    rw.append(pltpu.async_copy(
                        acc_sh.at[pl.ds(r0, 128)],
                        rbuf.at[pl.ds(j * 128, 128)], gsem))
                for w in rw:
                    w.wait()
                ww = []
                for j in range(jn):
                    r0 = s * RPT + (j0 + j) * 128
                    ww.append(pltpu.async_copy(
                        rbuf.at[pl.ds(j * 128, 128)],
                        out_hbm.at[pl.ds((k_core * NC + c) * Npad + r0,
                                         128)], gsem))
                for w in ww:
                    w.wait()
            return 0

        lax.fori_loop(0, NC, chunk_body, 0)

    return scat_kernel(yflat, src, dst, ew, dinv)


def _c1_body(z_ref, t_ref, out_ref, stats_ref, acc_ref):
    i = pl.program_id(0)
    nb = pl.num_programs(0)
    t = t_ref[...]
    NC = t.shape[1]
    o = z_ref[...] + jnp.concatenate(
        [t[0, cc] + t[1, cc] for cc in range(NC)], axis=1)
    out_ref[...] = o

    @pl.when(i == 0)
    def _():
        acc_ref[...] = jnp.zeros_like(acc_ref)

    s1 = jnp.sum(o, axis=0, keepdims=True)
    s2 = jnp.sum(o * o, axis=0, keepdims=True)
    acc_ref[...] += jnp.concatenate([s1, s2], axis=0)

    @pl.when(i == nb - 1)
    def _():
        stats_ref[...] = acc_ref[...]


def _run_c1(z, t, RB):
    N, F = z.shape
    NC = t.shape[1]
    CW = t.shape[3]
    grid = (N // RB,)
    return pl.pallas_call(
        _c1_body, grid=grid,
        in_specs=[
            pl.BlockSpec((RB, F), lambda i: (i, 0)),
            pl.BlockSpec((2, NC, RB, CW), lambda i: (0, 0, i, 0)),
        ],
        out_specs=[
            pl.BlockSpec((RB, F), lambda i: (i, 0)),
            pl.BlockSpec((2, F), lambda i: (0, 0)),
        ],
        out_shape=[
            jax.ShapeDtypeStruct((N, F), F32),
            jax.ShapeDtypeStruct((2, F), F32),
        ],
        scratch_shapes=[pltpu.VMEM((2, F), F32)],
    )(z, t)


def _c2_body(o_ref, st_ref, W0_ref, W1_ref, b_ref, y2_ref, z2_ref, *, n):
    st = st_ref[...]
    mu = st[0:1] / n
    var = st[1:2] / n - mu * mu
    h = _lrelu((o_ref[...] - mu) * lax.rsqrt(var + 1e-5))
    y2 = _dot(h, W1_ref[...])
    for cc in range(y2.shape[1] // 64):
        y2_ref[cc] = y2[:, cc * 64:(cc + 1) * 64]
    z2_ref[...] = _dot(h, W0_ref[...]) + b_ref[...]


def _run_c2(out1, stats, W0, W1, b, RB):
    N, F = out1.shape
    O = W0.shape[1]
    NC2 = O // 64
    grid = (N // RB,)
    return pl.pallas_call(
        functools.partial(_c2_body, n=float(N)), grid=grid,
        in_specs=[
            pl.BlockSpec((RB, F), lambda i: (i, 0)),
            pl.BlockSpec((2, F), lambda i: (0, 0)),
            pl.BlockSpec(W0.shape, lambda i: (0, 0)),
            pl.BlockSpec(W1.shape, lambda i: (0, 0)),
            pl.BlockSpec((1, O), lambda i: (0, 0)),
        ],
        out_specs=[
            pl.BlockSpec((NC2, RB, 64), lambda i: (0, i, 0)),
            pl.BlockSpec((RB, O), lambda i: (i, 0)),
        ],
        out_shape=[
            jax.ShapeDtypeStruct((NC2, N, 64), F32),
            jax.ShapeDtypeStruct((N, O), F32),
        ],
    )(out1, stats, W0, W1, b)


def _d2_body(o_ref, st_ref, out_ref, *, n):
    st = st_ref[...]
    mu = st[0:1] / n
    var = st[1:2] / n - mu * mu
    out_ref[...] = _lrelu((o_ref[...] - mu) * lax.rsqrt(var + 1e-5))


def _run_d2(out2, stats, RB):
    N, F = out2.shape
    grid = (N // RB,)
    return pl.pallas_call(
        functools.partial(_d2_body, n=float(N)), grid=grid,
        in_specs=[
            pl.BlockSpec((RB, F), lambda i: (i, 0)),
            pl.BlockSpec((2, F), lambda i: (0, 0)),
        ],
        out_specs=pl.BlockSpec((RB, F), lambda i: (i, 0)),
        out_shape=jax.ShapeDtypeStruct((N, F), F32),
    )(out2, stats)


def kernel(x, latent_vector, style_vector, edge_index, edge_attr, batch_size,
           nroi, W_fc1, b_fc1, Ws_fc1, bs_fc1, W_fc2, b_fc2, Ws_fc2, bs_fc2,
           W_fc3, b_fc3, Ws_fc3, bs_fc3, W_fc4, b_fc4, Ws_fc4, bs_fc4,
           W0_g1, W1_g1, b_g1, W0_g2, W1_g2, b_g2):
    N, IN = x.shape
    B, LAT = latent_vector.shape
    R = N // B
    E = edge_index.shape[1]
    F2 = W_fc4.shape[1]
    O = W0_g2.shape[1]
    src = edge_index[0]
    dst = edge_index[1]

    degp = _sc_deg(dst, edge_attr, N, E).reshape(2, B, R)

    Ws = [W_fc1, b_fc1.reshape(1, -1), Ws_fc1, bs_fc1.reshape(1, -1),
          W_fc2, b_fc2.reshape(1, -1), Ws_fc2, bs_fc2.reshape(1, -1),
          W_fc3, b_fc3.reshape(1, -1), Ws_fc3, bs_fc3.reshape(1, -1),
          W_fc4, b_fc4.reshape(1, -1), Ws_fc4, bs_fc4.reshape(1, -1),
          W0_g1, W1_g1, b_g1.reshape(1, -1)]
    y1, z1, dinv = _run_a(x, latent_vector, style_vector, degp, Ws, F2)
    dinv = dinv.reshape(N)

    Npad = -(-N // 2048) * 2048

    def unpad(t, NC):
        return t.reshape(2, NC, Npad, 64)[:, :, :N]

    NC1 = F2 // 64
    NC2 = O // 64
    t1 = unpad(_sc_edge_scatter(y1.reshape(NC1 * N, 64), src, dst,
                                edge_attr, dinv, N, E, NC1), NC1)
    out1, stats1 = _run_c1(z1, t1, 1000)
    y2, z2 = _run_c2(out1, stats1, W0_g2, W1_g2, b_g2.reshape(1, -1), 1000)

    t2 = unpad(_sc_edge_scatter(y2.reshape(NC2 * N, 64), src, dst,
                                edge_attr, dinv, N, E, NC2), NC2)
    out2, stats2 = _run_c1(z2, t2, 1000)
    h = _run_d2(out2, stats2, 1000)
    return h.reshape(B, R, O)


# speedup vs baseline: 9.1705x; 1.0278x over previous
import functools

import jax
import jax.numpy as jnp
from jax import lax
from jax.experimental import pallas as pl
from jax.experimental.pallas import tpu as pltpu
from jax.experimental.pallas import tpu_sc as plsc

F32 = jnp.float32
_HIGH = lax.Precision.HIGHEST


def _dot(a, b):
    return jnp.dot(a, b, preferred_element_type=F32)


def _lrelu(x):
    return jnp.where(x >= 0, x, 0.2 * x)


def _a_body(x_ref, lat_ref, sty_ref, degp_ref,
            W1_ref, b1_ref, Ws1_ref, bs1_ref,
            W2_ref, b2_ref, Ws2_ref, bs2_ref,
            W3_ref, b3_ref, Ws3_ref, bs3_ref,
            W4_ref, b4_ref, Ws4_ref, bs4_ref,
            W0g_ref, W1g_ref, bg_ref,
            y1_ref, z1_ref, dinv_ref):
    xb = x_ref[...]
    R = xb.shape[0]
    lv = lat_ref[0]
    h = jnp.concatenate([xb, jnp.broadcast_to(lv, (R, lv.shape[1]))], axis=1)
    sv = sty_ref[0]

    def unit(h, W_r, b_r, Ws_r, bs_r):
        t = _dot(h, W_r[...]) + b_r[...]
        mu = jnp.mean(t, axis=0, keepdims=True)
        d = t - mu
        var = jnp.mean(d * d, axis=0, keepdims=True)
        s = _dot(sv, Ws_r[...]) + bs_r[...]
        F = t.shape[1]
        gamma = s[:, :F]
        beta = s[:, F:]
        out = d * lax.rsqrt(var + 1e-5) * (1.0 + gamma) + beta
        return _lrelu(out)

    h = unit(h, W1_ref, b1_ref, Ws1_ref, bs1_ref)
    h = unit(h, W2_ref, b2_ref, Ws2_ref, bs2_ref)
    h = unit(h, W3_ref, b3_ref, Ws3_ref, bs3_ref)
    h = unit(h, W4_ref, b4_ref, Ws4_ref, bs4_ref)

    y1 = _dot(h, W1g_ref[...])
    for cc in range(y1.shape[1] // 64):
        y1_ref[cc] = y1[:, cc * 64:(cc + 1) * 64]
    z1_ref[...] = _dot(h, W0g_ref[...]) + bg_ref[...]

    degb = degp_ref[0, 0] + degp_ref[1, 0]
    dinv_ref[0] = jnp.where(degb > 0, 1.0 / jnp.sqrt(degb + 1e-12), 0.0)


def _run_a(x, lat, sty, degp, Ws, F2):
    N, IN = x.shape
    B, LAT = lat.shape
    R = N // B
    S = sty.shape[1]
    NC = F2 // 64
    full = lambda shape: pl.BlockSpec(shape, lambda b: tuple(0 for _ in shape))
    in_specs = [
        pl.BlockSpec((R, IN), lambda b: (b, 0)),
        pl.BlockSpec((1, 1, LAT), lambda b: (b, 0, 0)),
        pl.BlockSpec((1, 1, S), lambda b: (b, 0, 0)),
        pl.BlockSpec((2, 1, 1, R), lambda b: (0, b, 0, 0)),
    ] + [full(w.shape) for w in Ws]
    out_shapes = [
        jax.ShapeDtypeStruct((NC, N, 64), F32),
        jax.ShapeDtypeStruct((N, F2), F32),
        jax.ShapeDtypeStruct((B, 1, R), F32),
    ]
    out_specs = [
        pl.BlockSpec((NC, R, 64), lambda b: (0, b, 0)),
        pl.BlockSpec((R, F2), lambda b: (b, 0)),
        pl.BlockSpec((1, 1, R), lambda b: (b, 0, 0)),
    ]
    return pl.pallas_call(
        _a_body, grid=(B,), in_specs=in_specs, out_specs=out_specs,
        out_shape=out_shapes,
    )(x, lat.reshape(B, 1, LAT), sty.reshape(B, 1, S),
      degp.reshape(2, B, 1, R), *Ws)


def _lane_bcast(vec, lane):
    idx = jnp.full((16, 1), lane, jnp.int32)
    dnums = lax.GatherDimensionNumbers(
        offset_dims=(), collapsed_slice_dims=(0,), start_index_map=(0,))
    return lax.gather(vec, idx, dnums, (1,),
                      mode=lax.GatherScatterMode.PROMISE_IN_BOUNDS)


def _zero2d(ref, rows, cols):
    z = jnp.zeros((16,), F32)

    def body(r, _):
        for v in range(cols // 16):
            ref[r, pl.ds(v * 16, 16)] = z
        return 0

    lax.fori_loop(0, rows, body, 0)


def _sc_deg(dst, ew, N, E):
    NBLK_SC = E // 256
    per_tile = -(-NBLK_SC // 16)
    mesh = plsc.VectorSubcoreMesh(core_axis_name="c", subcore_axis_name="s")

    @functools.partial(
        pl.kernel, mesh=mesh,
        out_type=jax.ShapeDtypeStruct((2 * N,), F32),
        scratch_types=[
            pltpu.VMEM((1, 128), jnp.int32),
            pltpu.VMEM((128,), F32),
            pltpu.VMEM((N,), F32),
            pltpu.VMEM_SHARED((N,), F32),
        ],
        compiler_params=pltpu.CompilerParams(needs_layout_passes=False),
    )
    def deg_kernel(dst_hbm, ew_hbm, out_hbm, idx_v, val_v, zb_v, acc_sh):
        c = lax.axis_index("c")
        s = lax.axis_index("s")
        z = jnp.zeros((16,), F32)

        def zbody(i, _):
            zb_v[pl.ds(i * 16, 16)] = z
            return 0

        lax.fori_loop(0, N // 16, zbody, 0)

        @pl.when(s == 0)
        def _():
            pltpu.sync_copy(zb_v, acc_sh)

        plsc.subcore_barrier()

        def step(i, _):
            blk = c * NBLK_SC + s + 16 * i

            @pl.when(s + 16 * i < NBLK_SC)
            def _():
                base = blk * 128
                pltpu.sync_copy(dst_hbm.at[pl.ds(base, 128)], idx_v.at[0])
                pltpu.sync_copy(ew_hbm.at[pl.ds(base, 128)], val_v)
                pltpu.sync_copy(val_v, acc_sh.at[idx_v.at[0]], add=True)
            return 0

        lax.fori_loop(0, per_tile, step, 0)
        plsc.subcore_barrier()

        @pl.when(s == 0)
        def _():
            pltpu.sync_copy(acc_sh, zb_v)
            pltpu.sync_copy(zb_v, out_hbm.at[pl.ds(c * N, N)])

    return deg_kernel(dst, ew)


def _sc_edge_scatter(yflat, src, dst, ew, dinv, N, E, NC):
    NBLK_SC = E // 256
    per_tile = -(-NBLK_SC // 16)
    Npad = -(-N // 2048) * 2048
    RPT = Npad // 16
    mesh = plsc.VectorSubcoreMesh(core_axis_name="c", subcore_axis_name="s")

    SB = 4
    SG = -(-per_tile // SB)
    SG2 = -(-SG // 2)
    CW = 64
    NBUF = 2 * SB

    @functools.partial(
        pl.kernel, mesh=mesh,
        out_type=jax.ShapeDtypeStruct((2 * NC * Npad, CW), F32),
        scratch_types=[
            pltpu.VMEM((N,), F32),
            pltpu.VMEM((NBUF, 128), jnp.int32),
            pltpu.VMEM((NBUF, 128), jnp.int32),
            pltpu.VMEM((NBUF, 128), jnp.int32),
            pltpu.VMEM((NBUF, 128), jnp.int32),
            pltpu.VMEM((NBUF, 128), F32),
            pltpu.VMEM((NBUF, 128), F32),
            pltpu.VMEM((NBUF * 128, CW), F32),
            pltpu.VMEM((128, CW), F32),
            pltpu.VMEM_SHARED((Npad, CW), F32),
            pltpu.SemaphoreType.DMA,
            pltpu.SemaphoreType.DMA,
            pltpu.SemaphoreType.DMA,
        ],
        compiler_params=pltpu.CompilerParams(
            needs_layout_passes=False, use_tc_tiling_on_sc=False),
    )
    def scat_kernel(y_hbm, src_hbm, dst_hbm, ew_hbm, dinv_hbm, out_hbm,
                    dinv_v, src2_v, dst2_v, sdst_v, adj_v, ew_v, norm_v,
                    rbuf, zbuf, acc_sh, esem, gsem, ssem):
        k_core = lax.axis_index("c")
        s = lax.axis_index("s")
        _zero2d(zbuf, 128, CW)
        pltpu.sync_copy(dinv_hbm, dinv_v)
        cnt = lax.div(NBLK_SC - 1 - s, 16) + 1

        def fire_stage(g, p):
            for b in range(SB):
                i_eff = jnp.minimum(g * SB + b, cnt - 1)
                base = (k_core * NBLK_SC + s + 16 * i_eff) * 128
                r = p * SB + b
                pltpu.async_copy(src_hbm.at[pl.ds(base, 128)],
                                 src2_v.at[r], esem)
                pltpu.async_copy(dst_hbm.at[pl.ds(base, 128)],
                                 dst2_v.at[r], esem)
                pltpu.async_copy(ew_hbm.at[pl.ds(base, 128)],
                                 ew_v.at[r], esem)

        def drain_stage(p):
            for b in range(SB):
                r = p * SB + b
                pltpu.make_async_copy(src_hbm.at[pl.ds(0, 128)],
                                      src2_v.at[r], esem).wait()
                pltpu.make_async_copy(dst_hbm.at[pl.ds(0, 128)],
                                      dst2_v.at[r], esem).wait()
                pltpu.make_async_copy(ew_hbm.at[pl.ds(0, 128)],
                                      ew_v.at[r], esem).wait()

        def norms(g, p, c):
            for b in range(SB):
                r = p * SB + b
                valid = (g * SB + b) < cnt
                for k in range(8):
                    sl = pl.ds(k * 16, 16)
                    sv = src2_v[r, sl]
                    dv = dst2_v[r, sl]
                    nv = -(ew_v[r, sl]
                           * plsc.load_gather(dinv_v, [sv])
                           * plsc.load_gather(dinv_v, [dv]))
                    norm_v[r, sl] = jnp.where(valid, nv,
                                              jnp.zeros((16,), F32))
                    adj_v[r, sl] = sv + c * N

        def fire_gather(p):
            for b in range(SB):
                r = p * SB + b
                pltpu.async_copy(y_hbm.at[adj_v.at[r]],
                                 rbuf.at[pl.ds(r * 128, 128)], gsem)

        def drain_gather(p):
            for b in range(SB):
                r = p * SB + b
                pltpu.make_async_copy(y_hbm.at[adj_v.at[r]],
                                      rbuf.at[pl.ds(r * 128, 128)],
                                      gsem).wait()

        def scale(p):
            for b in range(SB):
                r = p * SB + b
                for k in range(8):
                    nv16 = norm_v[r, pl.ds(k * 16, 16)]

                    def body(lane, _):
                        nrm = _lane_bcast(nv16, lane)
                        e = r * 128 + k * 16 + lane
                        for v in range(CW // 16):
                            cs = pl.ds(v * 16, 16)
                            rbuf[e, cs] = rbuf[e, cs] * nrm
                        return 0

                    lax.fori_loop(0, 16, body, 0)

        def snap_and_fire_scatter(p):
            for b in range(SB):
                r = p * SB + b
                for k in range(8):
                    sl = pl.ds(k * 16, 16)
                    sdst_v[r, sl] = dst2_v[r, sl]
            for b in range(SB):
                r = p * SB + b
                pltpu.async_copy(rbuf.at[pl.ds(r * 128, 128)],
                                 acc_sh.at[sdst_v.at[r]], ssem, add=True)

        def drain_scatter(p):
            for b in range(SB):
                r = p * SB + b
                pltpu.make_async_copy(rbuf.at[pl.ds(r * 128, 128)],
                                      acc_sh.at[sdst_v.at[r]], ssem).wait()

        def chunk_body(c, _):
            zw = []
            for j in range(RPT // 128):
                zw.append(pltpu.async_copy(
                    zbuf, acc_sh.at[pl.ds(s * RPT + j * 128, 128)], ssem))
            for w in zw:
                w.wait()
            plsc.subcore_barrier()

            fire_stage(0, 0)
            drain_stage(0)
            norms(0, 0, c)
            fire_gather(0)
            fire_stage(1, 1)

            def step(t, _):
                g0 = 2 * t
                g1 = 2 * t + 1
                drain_stage(1)
                norms(g1, 1, c)
                drain_gather(0)
                fire_gather(1)
                scale(0)
                snap_and_fire_scatter(0)
                fire_stage(g0 + 2, 0)
                drain_stage(0)
                norms(g0 + 2, 0, c)
                drain_gather(1)
                scale(1)
                snap_and_fire_scatter(1)
                drain_scatter(0)
                fire_gather(0)
                fire_stage(g1 + 2, 1)
                drain_scatter(1)
                return 0

            lax.fori_loop(0, SG2, step, 0)
            drain_gather(0)
            drain_stage(1)
            plsc.subcore_barrier()
            for j0 in range(0, RPT // 128, 4):
                jn = min(4, RPT // 128 - j0)
                rw = []
                for j in range(jn):
                    r0 = s * RPT + (j0 + j) * 128
                    rw.append(pltpu.async_copy(
                        acc_sh.at[pl.ds(r0, 128)],
                        rbuf.at[pl.ds(j * 128, 128)], gsem))
                for w in rw:
                    w.wait()
                ww = []
                for j in range(jn):
                    r0 = s * RPT + (j0 + j) * 128
                    ww.append(pltpu.async_copy(
                        rbuf.at[pl.ds(j * 128, 128)],
                        out_hbm.at[pl.ds((k_core * NC + c) * Npad + r0,
                                         128)], gsem))
                for w in ww:
                    w.wait()
            return 0

        lax.fori_loop(0, NC, chunk_body, 0)

    return scat_kernel(yflat, src, dst, ew, dinv)


def _c1_body(z_ref, t_ref, out_ref, stats_ref, acc_ref):
    i = pl.program_id(0)
    nb = pl.num_programs(0)
    t = t_ref[...]
    NC = t.shape[1]
    o = z_ref[...] + jnp.concatenate(
        [t[0, cc] + t[1, cc] for cc in range(NC)], axis=1)
    out_ref[...] = o

    @pl.when(i == 0)
    def _():
        acc_ref[...] = jnp.zeros_like(acc_ref)

    s1 = jnp.sum(o, axis=0, keepdims=True)
    s2 = jnp.sum(o * o, axis=0, keepdims=True)
    acc_ref[...] += jnp.concatenate([s1, s2], axis=0)

    @pl.when(i == nb - 1)
    def _():
        stats_ref[...] = acc_ref[...]


def _run_c1(z, t, RB):
    N, F = z.shape
    NC = t.shape[1]
    CW = t.shape[3]
    grid = (N // RB,)
    return pl.pallas_call(
        _c1_body, grid=grid,
        in_specs=[
            pl.BlockSpec((RB, F), lambda i: (i, 0)),
            pl.BlockSpec((2, NC, RB, CW), lambda i: (0, 0, i, 0)),
        ],
        out_specs=[
            pl.BlockSpec((RB, F), lambda i: (i, 0)),
            pl.BlockSpec((2, F), lambda i: (0, 0)),
        ],
        out_shape=[
            jax.ShapeDtypeStruct((N, F), F32),
            jax.ShapeDtypeStruct((2, F), F32),
        ],
        scratch_shapes=[pltpu.VMEM((2, F), F32)],
    )(z, t)


def _c2_body(o_ref, st_ref, W0_ref, W1_ref, b_ref, y2_ref, z2_ref, *, n):
    st = st_ref[...]
    mu = st[0:1] / n
    var = st[1:2] / n - mu * mu
    h = _lrelu((o_ref[...] - mu) * lax.rsqrt(var + 1e-5))
    y2 = _dot(h, W1_ref[...])
    for cc in range(y2.shape[1] // 64):
        y2_ref[cc] = y2[:, cc * 64:(cc + 1) * 64]
    z2_ref[...] = _dot(h, W0_ref[...]) + b_ref[...]


def _run_c2(out1, stats, W0, W1, b, RB):
    N, F = out1.shape
    O = W0.shape[1]
    NC2 = O // 64
    grid = (N // RB,)
    return pl.pallas_call(
        functools.partial(_c2_body, n=float(N)), grid=grid,
        in_specs=[
            pl.BlockSpec((RB, F), lambda i: (i, 0)),
            pl.BlockSpec((2, F), lambda i: (0, 0)),
            pl.BlockSpec(W0.shape, lambda i: (0, 0)),
            pl.BlockSpec(W1.shape, lambda i: (0, 0)),
            pl.BlockSpec((1, O), lambda i: (0, 0)),
        ],
        out_specs=[
            pl.BlockSpec((NC2, RB, 64), lambda i: (0, i, 0)),
            pl.BlockSpec((RB, O), lambda i: (i, 0)),
        ],
        out_shape=[
            jax.ShapeDtypeStruct((NC2, N, 64), F32),
            jax.ShapeDtypeStruct((N, O), F32),
        ],
    )(out1, stats, W0, W1, b)


def _d2_body(o_ref, st_ref, out_ref, *, n):
    st = st_ref[...]
    mu = st[0:1] / n
    var = st[1:2] / n - mu * mu
    out_ref[...] = _lrelu((o_ref[...] - mu) * lax.rsqrt(var + 1e-5))


def _run_d2(out2, stats, RB):
    N, F = out2.shape
    grid = (N // RB,)
    return pl.pallas_call(
        functools.partial(_d2_body, n=float(N)), grid=grid,
        in_specs=[
            pl.BlockSpec((RB, F), lambda i: (i, 0)),
            pl.BlockSpec((2, F), lambda i: (0, 0)),
        ],
        out_specs=pl.BlockSpec((RB, F), lambda i: (i, 0)),
        out_shape=jax.ShapeDtypeStruct((N, F), F32),
    )(out2, stats)


def kernel(x, latent_vector, style_vector, edge_index, edge_attr, batch_size,
           nroi, W_fc1, b_fc1, Ws_fc1, bs_fc1, W_fc2, b_fc2, Ws_fc2, bs_fc2,
           W_fc3, b_fc3, Ws_fc3, bs_fc3, W_fc4, b_fc4, Ws_fc4, bs_fc4,
           W0_g1, W1_g1, b_g1, W0_g2, W1_g2, b_g2):
    N, IN = x.shape
    B, LAT = latent_vector.shape
    R = N // B
    E = edge_index.shape[1]
    F2 = W_fc4.shape[1]
    O = W0_g2.shape[1]
    src = edge_index[0]
    dst = edge_index[1]

    degp = _sc_deg(dst, edge_attr, N, E).reshape(2, B, R)

    Ws = [W_fc1, b_fc1.reshape(1, -1), Ws_fc1, bs_fc1.reshape(1, -1),
          W_fc2, b_fc2.reshape(1, -1), Ws_fc2, bs_fc2.reshape(1, -1),
          W_fc3, b_fc3.reshape(1, -1), Ws_fc3, bs_fc3.reshape(1, -1),
          W_fc4, b_fc4.reshape(1, -1), Ws_fc4, bs_fc4.reshape(1, -1),
          W0_g1, W1_g1, b_g1.reshape(1, -1)]
    y1, z1, dinv = _run_a(x, latent_vector, style_vector, degp, Ws, F2)
    dinv = dinv.reshape(N)

    Npad = -(-N // 2048) * 2048

    def unpad(t, NC):
        return t.reshape(2, NC, Npad, 64)[:, :, :N]

    NC1 = F2 // 64
    NC2 = O // 64
    t1 = unpad(_sc_edge_scatter(y1.reshape(NC1 * N, 64), src, dst,
                                edge_attr, dinv, N, E, NC1), NC1)
    out1, stats1 = _run_c1(z1, t1, 1000)
    y2, z2 = _run_c2(out1, stats1, W0_g2, W1_g2, b_g2.reshape(1, -1), 1000)

    t2 = unpad(_sc_edge_scatter(y2.reshape(NC2 * N, 64), src, dst,
                                edge_attr, dinv, N, E, NC2), NC2)
    out2, stats2 = _run_c1(z2, t2, 1000)
    h = _run_d2(out2, stats2, 1000)
    return h.reshape(B, R, O)


# speedup vs baseline: 9.8356x; 1.0725x over previous
import functools

import jax
import jax.numpy as jnp
from jax import lax
from jax.experimental import pallas as pl
from jax.experimental.pallas import tpu as pltpu
from jax.experimental.pallas import tpu_sc as plsc

F32 = jnp.float32
_HIGH = lax.Precision.HIGHEST


def _dot(a, b):
    return jnp.dot(a, b, preferred_element_type=F32)


def _lrelu(x):
    return jnp.where(x >= 0, x, 0.2 * x)


def _a_body(x_ref, lat_ref, sty_ref,
            W1_ref, b1_ref, Ws1_ref, bs1_ref,
            W2_ref, b2_ref, Ws2_ref, bs2_ref,
            W3_ref, b3_ref, Ws3_ref, bs3_ref,
            W4_ref, b4_ref, Ws4_ref, bs4_ref,
            W0g_ref, W1g_ref, bg_ref,
            y1_ref, z1_ref):
    xb = x_ref[...]
    R = xb.shape[0]
    lv = lat_ref[0]
    h = jnp.concatenate([xb, jnp.broadcast_to(lv, (R, lv.shape[1]))], axis=1)
    sv = sty_ref[0]

    def unit(h, W_r, b_r, Ws_r, bs_r):
        t = _dot(h, W_r[...]) + b_r[...]
        mu = jnp.mean(t, axis=0, keepdims=True)
        d = t - mu
        var = jnp.mean(d * d, axis=0, keepdims=True)
        s = _dot(sv, Ws_r[...]) + bs_r[...]
        F = t.shape[1]
        gamma = s[:, :F]
        beta = s[:, F:]
        out = d * lax.rsqrt(var + 1e-5) * (1.0 + gamma) + beta
        return _lrelu(out)

    h = unit(h, W1_ref, b1_ref, Ws1_ref, bs1_ref)
    h = unit(h, W2_ref, b2_ref, Ws2_ref, bs2_ref)
    h = unit(h, W3_ref, b3_ref, Ws3_ref, bs3_ref)
    h = unit(h, W4_ref, b4_ref, Ws4_ref, bs4_ref)

    y1 = _dot(h, W1g_ref[...])
    for cc in range(y1.shape[1] // 64):
        y1_ref[cc] = y1[:, cc * 64:(cc + 1) * 64]
    z1_ref[...] = _dot(h, W0g_ref[...]) + bg_ref[...]


def _dinv_body(degp_ref, dinv_ref):
    degb = degp_ref[0:1] + degp_ref[1:2]
    dinv_ref[...] = jnp.where(degb > 0, 1.0 / jnp.sqrt(degb + 1e-12), 0.0)


def _run_dinv(degp):
    _, N = degp.shape
    return pl.pallas_call(
        _dinv_body,
        out_shape=jax.ShapeDtypeStruct((1, N), F32),
    )(degp)


def _run_a(x, lat, sty, Ws, F2):
    N, IN = x.shape
    B, LAT = lat.shape
    R = N // B
    S = sty.shape[1]
    NC = F2 // 64
    full = lambda shape: pl.BlockSpec(shape, lambda b: tuple(0 for _ in shape))
    in_specs = [
        pl.BlockSpec((R, IN), lambda b: (b, 0)),
        pl.BlockSpec((1, 1, LAT), lambda b: (b, 0, 0)),
        pl.BlockSpec((1, 1, S), lambda b: (b, 0, 0)),
    ] + [full(w.shape) for w in Ws]
    out_shapes = [
        jax.ShapeDtypeStruct((NC, N, 64), F32),
        jax.ShapeDtypeStruct((N, F2), F32),
    ]
    out_specs = [
        pl.BlockSpec((NC, R, 64), lambda b: (0, b, 0)),
        pl.BlockSpec((R, F2), lambda b: (b, 0)),
    ]
    return pl.pallas_call(
        _a_body, grid=(B,), in_specs=in_specs, out_specs=out_specs,
        out_shape=out_shapes,
    )(x, lat.reshape(B, 1, LAT), sty.reshape(B, 1, S), *Ws)


def _lane_bcast(vec, lane):
    idx = jnp.full((16, 1), lane, jnp.int32)
    dnums = lax.GatherDimensionNumbers(
        offset_dims=(), collapsed_slice_dims=(0,), start_index_map=(0,))
    return lax.gather(vec, idx, dnums, (1,),
                      mode=lax.GatherScatterMode.PROMISE_IN_BOUNDS)


def _zero2d(ref, rows, cols):
    z = jnp.zeros((16,), F32)

    def body(r, _):
        for v in range(cols // 16):
            ref[r, pl.ds(v * 16, 16)] = z
        return 0

    lax.fori_loop(0, rows, body, 0)


def _sc_deg(dst, ew, N, E):
    NBLK_SC = E // 256
    per_tile = -(-NBLK_SC // 16)
    mesh = plsc.VectorSubcoreMesh(core_axis_name="c", subcore_axis_name="s")

    @functools.partial(
        pl.kernel, mesh=mesh,
        out_type=jax.ShapeDtypeStruct((2 * N,), F32),
        scratch_types=[
            pltpu.VMEM((1, 128), jnp.int32),
            pltpu.VMEM((128,), F32),
            pltpu.VMEM((N,), F32),
            pltpu.VMEM_SHARED((N,), F32),
        ],
        compiler_params=pltpu.CompilerParams(needs_layout_passes=False),
    )
    def deg_kernel(dst_hbm, ew_hbm, out_hbm, idx_v, val_v, zb_v, acc_sh):
        c = lax.axis_index("c")
        s = lax.axis_index("s")
        z = jnp.zeros((16,), F32)

        def zbody(i, _):
            zb_v[pl.ds(i * 16, 16)] = z
            return 0

        lax.fori_loop(0, N // 16, zbody, 0)

        @pl.when(s == 0)
        def _():
            pltpu.sync_copy(zb_v, acc_sh)

        plsc.subcore_barrier()

        def step(i, _):
            blk = c * NBLK_SC + s + 16 * i

            @pl.when(s + 16 * i < NBLK_SC)
            def _():
                base = blk * 128
                pltpu.sync_copy(dst_hbm.at[pl.ds(base, 128)], idx_v.at[0])
                pltpu.sync_copy(ew_hbm.at[pl.ds(base, 128)], val_v)
                pltpu.sync_copy(val_v, acc_sh.at[idx_v.at[0]], add=True)
            return 0

        lax.fori_loop(0, per_tile, step, 0)
        plsc.subcore_barrier()

        @pl.when(s == 0)
        def _():
            pltpu.sync_copy(acc_sh, zb_v)
            pltpu.sync_copy(zb_v, out_hbm.at[pl.ds(c * N, N)])

    return deg_kernel(dst, ew)


def _sc_edge_scatter(yflat, src, dst, ew, dinv, N, E, NC):
    NBLK_SC = E // 256
    per_tile = -(-NBLK_SC // 16)
    Npad = -(-N // 2048) * 2048
    RPT = Npad // 16
    mesh = plsc.VectorSubcoreMesh(core_axis_name="c", subcore_axis_name="s")

    SB = 4
    SG = -(-per_tile // SB)
    SG2 = -(-SG // 2)
    CW = 64
    NBUF = 2 * SB

    @functools.partial(
        pl.kernel, mesh=mesh,
        out_type=jax.ShapeDtypeStruct((2 * NC * Npad, CW), F32),
        scratch_types=[
            pltpu.VMEM((N,), F32),
            pltpu.VMEM((NBUF, 128), jnp.int32),
            pltpu.VMEM((NBUF, 128), jnp.int32),
            pltpu.VMEM((NBUF, 128), jnp.int32),
            pltpu.VMEM((NBUF, 128), jnp.int32),
            pltpu.VMEM((NBUF, 128), F32),
            pltpu.VMEM((NBUF, 128), F32),
            pltpu.VMEM((NBUF * 128, CW), F32),
            pltpu.VMEM((128, CW), F32),
            pltpu.VMEM_SHARED((Npad, CW), F32),
            pltpu.SemaphoreType.DMA,
            pltpu.SemaphoreType.DMA,
            pltpu.SemaphoreType.DMA,
        ],
        compiler_params=pltpu.CompilerParams(
            needs_layout_passes=False, use_tc_tiling_on_sc=False),
    )
    def scat_kernel(y_hbm, src_hbm, dst_hbm, ew_hbm, dinv_hbm, out_hbm,
                    dinv_v, src2_v, dst2_v, sdst_v, adj_v, ew_v, norm_v,
                    rbuf, zbuf, acc_sh, esem, gsem, ssem):
        k_core = lax.axis_index("c")
        s = lax.axis_index("s")
        _zero2d(zbuf, 128, CW)
        pltpu.sync_copy(dinv_hbm, dinv_v)
        cnt = lax.div(NBLK_SC - 1 - s, 16) + 1

        def fire_stage(g, p):
            for b in range(SB):
                i_eff = jnp.minimum(g * SB + b, cnt - 1)
                base = (k_core * NBLK_SC + s + 16 * i_eff) * 128
                r = p * SB + b
                pltpu.async_copy(src_hbm.at[pl.ds(base, 128)],
                                 src2_v.at[r], esem)
                pltpu.async_copy(dst_hbm.at[pl.ds(base, 128)],
                                 dst2_v.at[r], esem)
                pltpu.async_copy(ew_hbm.at[pl.ds(base, 128)],
                                 ew_v.at[r], esem)

        def drain_stage(p):
            for b in range(SB):
                r = p * SB + b
                pltpu.make_async_copy(src_hbm.at[pl.ds(0, 128)],
                                      src2_v.at[r], esem).wait()
                pltpu.make_async_copy(dst_hbm.at[pl.ds(0, 128)],
                                      dst2_v.at[r], esem).wait()
                pltpu.make_async_copy(ew_hbm.at[pl.ds(0, 128)],
                                      ew_v.at[r], esem).wait()

        def norms(g, p, c):
            for b in range(SB):
                r = p * SB + b
                valid = (g * SB + b) < cnt
                for k in range(8):
                    sl = pl.ds(k * 16, 16)
                    sv = src2_v[r, sl]
                    dv = dst2_v[r, sl]
                    nv = -(ew_v[r, sl]
                           * plsc.load_gather(dinv_v, [sv])
                           * plsc.load_gather(dinv_v, [dv]))
                    norm_v[r, sl] = jnp.where(valid, nv,
                                              jnp.zeros((16,), F32))
                    adj_v[r, sl] = sv + c * N

        def fire_gather(p):
            for b in range(SB):
                r = p * SB + b
                pltpu.async_copy(y_hbm.at[adj_v.at[r]],
                                 rbuf.at[pl.ds(r * 128, 128)], gsem)

        def drain_gather(p):
            for b in range(SB):
                r = p * SB + b
                pltpu.make_async_copy(y_hbm.at[adj_v.at[r]],
                                      rbuf.at[pl.ds(r * 128, 128)],
                                      gsem).wait()

        def scale(p):
            for b in range(SB):
                r = p * SB + b
                for k in range(8):
                    nv16 = norm_v[r, pl.ds(k * 16, 16)]

                    def body(lane, _):
                        nrm = _lane_bcast(nv16, lane)
                        e = r * 128 + k * 16 + lane
                        for v in range(CW // 16):
                            cs = pl.ds(v * 16, 16)
                            rbuf[e, cs] = rbuf[e, cs] * nrm
                        return 0

                    lax.fori_loop(0, 16, body, 0)

        def snap_and_fire_scatter(p):
            for b in range(SB):
                r = p * SB + b
                for k in range(8):
                    sl = pl.ds(k * 16, 16)
                    sdst_v[r, sl] = dst2_v[r, sl]
            for b in range(SB):
                r = p * SB + b
                pltpu.async_copy(rbuf.at[pl.ds(r * 128, 128)],
                                 acc_sh.at[sdst_v.at[r]], ssem, add=True)

        def drain_scatter(p):
            for b in range(SB):
                r = p * SB + b
                pltpu.make_async_copy(rbuf.at[pl.ds(r * 128, 128)],
                                      acc_sh.at[sdst_v.at[r]], ssem).wait()

        def chunk_body(c, _):
            zw = []
            for j in range(RPT // 128):
                zw.append(pltpu.async_copy(
                    zbuf, acc_sh.at[pl.ds(s * RPT + j * 128, 128)], ssem))
            for w in zw:
                w.wait()
            plsc.subcore_barrier()

            fire_stage(0, 0)
            drain_stage(0)
            norms(0, 0, c)
            fire_gather(0)
            fire_stage(1, 1)

            def step(t, _):
                g0 = 2 * t
                g1 = 2 * t + 1
                drain_stage(1)
                norms(g1, 1, c)
                drain_gather(0)
                fire_gather(1)
                scale(0)
                snap_and_fire_scatter(0)
                fire_stage(g0 + 2, 0)
                drain_stage(0)
                norms(g0 + 2, 0, c)
                drain_gather(1)
                scale(1)
                snap_and_fire_scatter(1)
                drain_scatter(0)
                fire_gather(0)
                fire_stage(g1 + 2, 1)
                drain_scatter(1)
                return 0

            lax.fori_loop(0, SG2, step, 0)
            drain_gather(0)
            drain_stage(1)
            plsc.subcore_barrier()
            for j0 in range(0, RPT // 128, 4):
                jn = min(4, RPT // 128 - j0)
                rw = []
                for j in range(jn):
                    r0 = s * RPT + (j0 + j) * 128
                    rw.append(pltpu.async_copy(
                        acc_sh.at[pl.ds(r0, 128)],
                        rbuf.at[pl.ds(j * 128, 128)], gsem))
                for w in rw:
                    w.wait()
                ww = []
                for j in range(jn):
                    r0 = s * RPT + (j0 + j) * 128
                    ww.append(pltpu.async_copy(
                        rbuf.at[pl.ds(j * 128, 128)],
                        out_hbm.at[pl.ds((k_core * NC + c) * Npad + r0,
                                         128)], gsem))
                for w in ww:
                    w.wait()
            return 0

        lax.fori_loop(0, NC, chunk_body, 0)

    return scat_kernel(yflat, src, dst, ew, dinv)


def _c1_body(z_ref, t_ref, out_ref, stats_ref, acc_ref):
    i = pl.program_id(0)
    nb = pl.num_programs(0)
    t = t_ref[...]
    NC = t.shape[1]
    o = z_ref[...] + jnp.concatenate(
        [t[0, cc] + t[1, cc] for cc in range(NC)], axis=1)
    out_ref[...] = o

    @pl.when(i == 0)
    def _():
        acc_ref[...] = jnp.zeros_like(acc_ref)

    s1 = jnp.sum(o, axis=0, keepdims=True)
    s2 = jnp.sum(o * o, axis=0, keepdims=True)
    acc_ref[...] += jnp.concatenate([s1, s2], axis=0)

    @pl.when(i == nb - 1)
    def _():
        stats_ref[...] = acc_ref[...]


def _run_c1(z, t, RB):
    N, F = z.shape
    NC = t.shape[1]
    CW = t.shape[3]
    grid = (N // RB,)
    return pl.pallas_call(
        _c1_body, grid=grid,
        in_specs=[
            pl.BlockSpec((RB, F), lambda i: (i, 0)),
            pl.BlockSpec((2, NC, RB, CW), lambda i: (0, 0, i, 0)),
        ],
        out_specs=[
            pl.BlockSpec((RB, F), lambda i: (i, 0)),
            pl.BlockSpec((2, F), lambda i: (0, 0)),
        ],
        out_shape=[
            jax.ShapeDtypeStruct((N, F), F32),
            jax.ShapeDtypeStruct((2, F), F32),
        ],
        scratch_shapes=[pltpu.VMEM((2, F), F32)],
    )(z, t)


def _c2_body(o_ref, st_ref, W0_ref, W1_ref, b_ref, y2_ref, z2_ref, *, n):
    st = st_ref[...]
    mu = st[0:1] / n
    var = st[1:2] / n - mu * mu
    h = _lrelu((o_ref[...] - mu) * lax.rsqrt(var + 1e-5))
    y2 = _dot(h, W1_ref[...])
    for cc in range(y2.shape[1] // 64):
        y2_ref[cc] = y2[:, cc * 64:(cc + 1) * 64]
    z2_ref[...] = _dot(h, W0_ref[...]) + b_ref[...]


def _run_c2(out1, stats, W0, W1, b, RB):
    N, F = out1.shape
    O = W0.shape[1]
    NC2 = O // 64
    grid = (N // RB,)
    return pl.pallas_call(
        functools.partial(_c2_body, n=float(N)), grid=grid,
        in_specs=[
            pl.BlockSpec((RB, F), lambda i: (i, 0)),
            pl.BlockSpec((2, F), lambda i: (0, 0)),
            pl.BlockSpec(W0.shape, lambda i: (0, 0)),
            pl.BlockSpec(W1.shape, lambda i: (0, 0)),
            pl.BlockSpec((1, O), lambda i: (0, 0)),
        ],
        out_specs=[
            pl.BlockSpec((NC2, RB, 64), lambda i: (0, i, 0)),
            pl.BlockSpec((RB, O), lambda i: (i, 0)),
        ],
        out_shape=[
            jax.ShapeDtypeStruct((NC2, N, 64), F32),
            jax.ShapeDtypeStruct((N, O), F32),
        ],
    )(out1, stats, W0, W1, b)


def _d2_body(o_ref, st_ref, out_ref, *, n):
    st = st_ref[...]
    mu = st[0:1] / n
    var = st[1:2] / n - mu * mu
    out_ref[...] = _lrelu((o_ref[...] - mu) * lax.rsqrt(var + 1e-5))


def _run_d2(out2, stats, RB):
    N, F = out2.shape
    grid = (N // RB,)
    return pl.pallas_call(
        functools.partial(_d2_body, n=float(N)), grid=grid,
        in_specs=[
            pl.BlockSpec((RB, F), lambda i: (i, 0)),
            pl.BlockSpec((2, F), lambda i: (0, 0)),
        ],
        out_specs=pl.BlockSpec((RB, F), lambda i: (i, 0)),
        out_shape=jax.ShapeDtypeStruct((N, F), F32),
    )(out2, stats)


def kernel(x, latent_vector, style_vector, edge_index, edge_attr, batch_size,
           nroi, W_fc1, b_fc1, Ws_fc1, bs_fc1, W_fc2, b_fc2, Ws_fc2, bs_fc2,
           W_fc3, b_fc3, Ws_fc3, bs_fc3, W_fc4, b_fc4, Ws_fc4, bs_fc4,
           W0_g1, W1_g1, b_g1, W0_g2, W1_g2, b_g2):
    N, IN = x.shape
    B, LAT = latent_vector.shape
    R = N // B
    E = edge_index.shape[1]
    F2 = W_fc4.shape[1]
    O = W0_g2.shape[1]
    src = edge_index[0]
    dst = edge_index[1]

    degp = _sc_deg(dst, edge_attr, N, E).reshape(2, N)

    Ws = [W_fc1, b_fc1.reshape(1, -1), Ws_fc1, bs_fc1.reshape(1, -1),
          W_fc2, b_fc2.reshape(1, -1), Ws_fc2, bs_fc2.reshape(1, -1),
          W_fc3, b_fc3.reshape(1, -1), Ws_fc3, bs_fc3.reshape(1, -1),
          W_fc4, b_fc4.reshape(1, -1), Ws_fc4, bs_fc4.reshape(1, -1),
          W0_g1, W1_g1, b_g1.reshape(1, -1)]
    y1, z1 = _run_a(x, latent_vector, style_vector, Ws, F2)
    dinv = _run_dinv(degp).reshape(N)

    Npad = -(-N // 2048) * 2048

    def unpad(t, NC):
        return t.reshape(2, NC, Npad, 64)[:, :, :N]

    NC1 = F2 // 64
    NC2 = O // 64
    t1 = unpad(_sc_edge_scatter(y1.reshape(NC1 * N, 64), src, dst,
                                edge_attr, dinv, N, E, NC1), NC1)
    out1, stats1 = _run_c1(z1, t1, 1000)
    y2, z2 = _run_c2(out1, stats1, W0_g2, W1_g2, b_g2.reshape(1, -1), 1000)

    t2 = unpad(_sc_edge_scatter(y2.reshape(NC2 * N, 64), src, dst,
                                edge_attr, dinv, N, E, NC2), NC2)
    out2, stats2 = _run_c1(z2, t2, 1000)
    h = _run_d2(out2, stats2, 1000)
    return h.reshape(B, R, O)


# speedup vs baseline: 9.9803x; 1.0147x over previous
import functools

import jax
import jax.numpy as jnp
from jax import lax
from jax.experimental import pallas as pl
from jax.experimental.pallas import tpu as pltpu
from jax.experimental.pallas import tpu_sc as plsc

F32 = jnp.float32
_HIGH = lax.Precision.HIGHEST


def _dot(a, b):
    return jnp.dot(a, b, preferred_element_type=F32)


def _lrelu(x):
    return jnp.where(x >= 0, x, 0.2 * x)


def _a_body(x_ref, lat_ref, sty_ref,
            W1_ref, b1_ref, Ws1_ref, bs1_ref,
            W2_ref, b2_ref, Ws2_ref, bs2_ref,
            W3_ref, b3_ref, Ws3_ref, bs3_ref,
            W4_ref, b4_ref, Ws4_ref, bs4_ref,
            W0g_ref, W1g_ref, bg_ref,
            y1_ref, z1_ref, *, GB, R):
    xb = x_ref[...]
    lv = lat_ref[0]
    lvr = jnp.broadcast_to(lv[:, None, :], (GB, R, lv.shape[1]))
    h = jnp.concatenate([xb, lvr.reshape(GB * R, lv.shape[1])], axis=1)
    sv = sty_ref[0]

    def unit(h, W_r, b_r, Ws_r, bs_r):
        t = _dot(h, W_r[...]) + b_r[...]
        F = t.shape[1]
        tr = t.reshape(GB, R, F)
        mu = jnp.mean(tr, axis=1, keepdims=True)
        d = tr - mu
        var = jnp.mean(d * d, axis=1, keepdims=True)
        s = _dot(sv, Ws_r[...]) + bs_r[...]
        gamma = s[:, None, :F]
        beta = s[:, None, F:]
        out = d * lax.rsqrt(var + 1e-5) * (1.0 + gamma) + beta
        return _lrelu(out.reshape(GB * R, F))

    h = unit(h, W1_ref, b1_ref, Ws1_ref, bs1_ref)
    h = unit(h, W2_ref, b2_ref, Ws2_ref, bs2_ref)
    h = unit(h, W3_ref, b3_ref, Ws3_ref, bs3_ref)
    h = unit(h, W4_ref, b4_ref, Ws4_ref, bs4_ref)

    y1 = _dot(h, W1g_ref[...])
    for cc in range(y1.shape[1] // 64):
        y1_ref[cc] = y1[:, cc * 64:(cc + 1) * 64]
    z1_ref[...] = _dot(h, W0g_ref[...]) + bg_ref[...]


def _dinv_body(degp_ref, dinv_ref):
    degb = degp_ref[0:1] + degp_ref[1:2]
    dinv_ref[...] = jnp.where(degb > 0, 1.0 / jnp.sqrt(degb + 1e-12), 0.0)


def _run_dinv(degp):
    _, N = degp.shape
    return pl.pallas_call(
        _dinv_body,
        out_shape=jax.ShapeDtypeStruct((1, N), F32),
    )(degp)


def _run_a(x, lat, sty, Ws, F2):
    N, IN = x.shape
    B, LAT = lat.shape
    R = N // B
    S = sty.shape[1]
    NC = F2 // 64
    GB = 5 if B % 5 == 0 else 1
    G = B // GB
    full = lambda shape: pl.BlockSpec(shape, lambda b: tuple(0 for _ in shape))
    in_specs = [
        pl.BlockSpec((GB * R, IN), lambda b: (b, 0)),
        pl.BlockSpec((1, GB, LAT), lambda b: (b, 0, 0)),
        pl.BlockSpec((1, GB, S), lambda b: (b, 0, 0)),
    ] + [full(w.shape) for w in Ws]
    out_shapes = [
        jax.ShapeDtypeStruct((NC, N, 64), F32),
        jax.ShapeDtypeStruct((N, F2), F32),
    ]
    out_specs = [
        pl.BlockSpec((NC, GB * R, 64), lambda b: (0, b, 0)),
        pl.BlockSpec((GB * R, F2), lambda b: (b, 0)),
    ]
    return pl.pallas_call(
        functools.partial(_a_body, GB=GB, R=R),
        grid=(G,), in_specs=in_specs, out_specs=out_specs,
        out_shape=out_shapes,
    )(x, lat.reshape(G, GB, LAT), sty.reshape(G, GB, S), *Ws)


def _lane_bcast(vec, lane):
    idx = jnp.full((16, 1), lane, jnp.int32)
    dnums = lax.GatherDimensionNumbers(
        offset_dims=(), collapsed_slice_dims=(0,), start_index_map=(0,))
    return lax.gather(vec, idx, dnums, (1,),
                      mode=lax.GatherScatterMode.PROMISE_IN_BOUNDS)


def _zero2d(ref, rows, cols):
    z = jnp.zeros((16,), F32)

    def body(r, _):
        for v in range(cols // 16):
            ref[r, pl.ds(v * 16, 16)] = z
        return 0

    lax.fori_loop(0, rows, body, 0)


def _sc_deg(dst, ew, N, E):
    NBLK_SC = E // 256
    per_tile = -(-NBLK_SC // 16)
    mesh = plsc.VectorSubcoreMesh(core_axis_name="c", subcore_axis_name="s")

    @functools.partial(
        pl.kernel, mesh=mesh,
        out_type=jax.ShapeDtypeStruct((2 * N,), F32),
        scratch_types=[
            pltpu.VMEM((1, 128), jnp.int32),
            pltpu.VMEM((128,), F32),
            pltpu.VMEM((N,), F32),
            pltpu.VMEM_SHARED((N,), F32),
        ],
        compiler_params=pltpu.CompilerParams(needs_layout_passes=False),
    )
    def deg_kernel(dst_hbm, ew_hbm, out_hbm, idx_v, val_v, zb_v, acc_sh):
        c = lax.axis_index("c")
        s = lax.axis_index("s")
        z = jnp.zeros((16,), F32)

        def zbody(i, _):
            zb_v[pl.ds(i * 16, 16)] = z
            return 0

        lax.fori_loop(0, N // 16, zbody, 0)

        @pl.when(s == 0)
        def _():
            pltpu.sync_copy(zb_v, acc_sh)

        plsc.subcore_barrier()

        def step(i, _):
            blk = c * NBLK_SC + s + 16 * i

            @pl.when(s + 16 * i < NBLK_SC)
            def _():
                base = blk * 128
                pltpu.sync_copy(dst_hbm.at[pl.ds(base, 128)], idx_v.at[0])
                pltpu.sync_copy(ew_hbm.at[pl.ds(base, 128)], val_v)
                pltpu.sync_copy(val_v, acc_sh.at[idx_v.at[0]], add=True)
            return 0

        lax.fori_loop(0, per_tile, step, 0)
        plsc.subcore_barrier()

        @pl.when(s == 0)
        def _():
            pltpu.sync_copy(acc_sh, zb_v)
            pltpu.sync_copy(zb_v, out_hbm.at[pl.ds(c * N, N)])

    return deg_kernel(dst, ew)


def _sc_edge_scatter(yflat, src, dst, ew, dinv, N, E, NC):
    NBLK_SC = E // 256
    per_tile = -(-NBLK_SC // 16)
    Npad = -(-N // 2048) * 2048
    RPT = Npad // 16
    mesh = plsc.VectorSubcoreMesh(core_axis_name="c", subcore_axis_name="s")

    SB = 4
    SG = -(-per_tile // SB)
    SG2 = -(-SG // 2)
    CW = 64
    NBUF = 2 * SB

    @functools.partial(
        pl.kernel, mesh=mesh,
        out_type=jax.ShapeDtypeStruct((2 * NC * Npad, CW), F32),
        scratch_types=[
            pltpu.VMEM((N,), F32),
            pltpu.VMEM((NBUF, 128), jnp.int32),
            pltpu.VMEM((NBUF, 128), jnp.int32),
            pltpu.VMEM((NBUF, 128), jnp.int32),
            pltpu.VMEM((NBUF, 128), jnp.int32),
            pltpu.VMEM((NBUF, 128), F32),
            pltpu.VMEM((NBUF, 128), F32),
            pltpu.VMEM((NBUF * 128, CW), F32),
            pltpu.VMEM((128, CW), F32),
            pltpu.VMEM_SHARED((Npad, CW), F32),
            pltpu.SemaphoreType.DMA,
            pltpu.SemaphoreType.DMA,
            pltpu.SemaphoreType.DMA,
        ],
        compiler_params=pltpu.CompilerParams(
            needs_layout_passes=False, use_tc_tiling_on_sc=False),
    )
    def scat_kernel(y_hbm, src_hbm, dst_hbm, ew_hbm, dinv_hbm, out_hbm,
                    dinv_v, src2_v, dst2_v, sdst_v, adj_v, ew_v, norm_v,
                    rbuf, zbuf, acc_sh, esem, gsem, ssem):
        k_core = lax.axis_index("c")
        s = lax.axis_index("s")
        _zero2d(zbuf, 128, CW)
        pltpu.sync_copy(dinv_hbm, dinv_v)
        cnt = lax.div(NBLK_SC - 1 - s, 16) + 1

        def fire_stage(g, p):
            for b in range(SB):
                i_eff = jnp.minimum(g * SB + b, cnt - 1)
                base = (k_core * NBLK_SC + s + 16 * i_eff) * 128
                r = p * SB + b
                pltpu.async_copy(src_hbm.at[pl.ds(base, 128)],
                                 src2_v.at[r], esem)
                pltpu.async_copy(dst_hbm.at[pl.ds(base, 128)],
                                 dst2_v.at[r], esem)
                pltpu.async_copy(ew_hbm.at[pl.ds(base, 128)],
                                 ew_v.at[r], esem)

        def drain_stage(p):
            for b in range(SB):
                r = p * SB + b
                pltpu.make_async_copy(src_hbm.at[pl.ds(0, 128)],
                                      src2_v.at[r], esem).wait()
                pltpu.make_async_copy(dst_hbm.at[pl.ds(0, 128)],
                                      dst2_v.at[r], esem).wait()
                pltpu.make_async_copy(ew_hbm.at[pl.ds(0, 128)],
                                      ew_v.at[r], esem).wait()

        def norms(g, p, c):
            for b in range(SB):
                r = p * SB + b
                valid = (g * SB + b) < cnt
                for k in range(8):
                    sl = pl.ds(k * 16, 16)
                    sv = src2_v[r, sl]
                    dv = dst2_v[r, sl]
                    nv = -(ew_v[r, sl]
                           * plsc.load_gather(dinv_v, [sv])
                           * plsc.load_gather(dinv_v, [dv]))
                    norm_v[r, sl] = jnp.where(valid, nv,
                                              jnp.zeros((16,), F32))
                    adj_v[r, sl] = sv + c * N

        def fire_gather(p):
            for b in range(SB):
                r = p * SB + b
                pltpu.async_copy(y_hbm.at[adj_v.at[r]],
                                 rbuf.at[pl.ds(r * 128, 128)], gsem)

        def drain_gather(p):
            for b in range(SB):
                r = p * SB + b
                pltpu.make_async_copy(y_hbm.at[adj_v.at[r]],
                                      rbuf.at[pl.ds(r * 128, 128)],
                                      gsem).wait()

        def scale(p):
            for b in range(SB):
                r = p * SB + b
                for k in range(8):
                    nv16 = norm_v[r, pl.ds(k * 16, 16)]

                    def body(lane, _):
                        nrm = _lane_bcast(nv16, lane)
                        e = r * 128 + k * 16 + lane
                        for v in range(CW // 16):
                            cs = pl.ds(v * 16, 16)
                            rbuf[e, cs] = rbuf[e, cs] * nrm
                        return 0

                    lax.fori_loop(0, 16, body, 0)

        def snap_and_fire_scatter(p):
            for b in range(SB):
                r = p * SB + b
                for k in range(8):
                    sl = pl.ds(k * 16, 16)
                    sdst_v[r, sl] = dst2_v[r, sl]
            for b in range(SB):
                r = p * SB + b
                pltpu.async_copy(rbuf.at[pl.ds(r * 128, 128)],
                                 acc_sh.at[sdst_v.at[r]], ssem, add=True)

        def drain_scatter(p):
            for b in range(SB):
                r = p * SB + b
                pltpu.make_async_copy(rbuf.at[pl.ds(r * 128, 128)],
                                      acc_sh.at[sdst_v.at[r]], ssem).wait()

        def chunk_body(c, _):
            zw = []
            for j in range(RPT // 128):
                zw.append(pltpu.async_copy(
                    zbuf, acc_sh.at[pl.ds(s * RPT + j * 128, 128)], ssem))
            for w in zw:
                w.wait()
            plsc.subcore_barrier()

            fire_stage(0, 0)
            drain_stage(0)
            norms(0, 0, c)
            fire_gather(0)
            fire_stage(1, 1)

            def step(t, _):
                g0 = 2 * t
                g1 = 2 * t + 1
                drain_stage(1)
                norms(g1, 1, c)
                drain_gather(0)
                fire_gather(1)
                scale(0)
                snap_and_fire_scatter(0)
                fire_stage(g0 + 2, 0)
                drain_stage(0)
                norms(g0 + 2, 0, c)
                drain_gather(1)
                scale(1)
                snap_and_fire_scatter(1)
                drain_scatter(0)
                fire_gather(0)
                fire_stage(g1 + 2, 1)
                drain_scatter(1)
                return 0

            lax.fori_loop(0, SG2, step, 0)
            drain_gather(0)
            drain_stage(1)
            plsc.subcore_barrier()
            for j0 in range(0, RPT // 128, 4):
                jn = min(4, RPT // 128 - j0)
                rw = []
                for j in range(jn):
                    r0 = s * RPT + (j0 + j) * 128
                    rw.append(pltpu.async_copy(
                        acc_sh.at[pl.ds(r0, 128)],
                        rbuf.at[pl.ds(j * 128, 128)], gsem))
                for w in rw:
                    w.wait()
                ww = []
                for j in range(jn):
                    r0 = s * RPT + (j0 + j) * 128
                    ww.append(pltpu.async_copy(
                        rbuf.at[pl.ds(j * 128, 128)],
                        out_hbm.at[pl.ds((k_core * NC + c) * Npad + r0,
                                         128)], gsem))
                for w in ww:
                    w.wait()
            return 0

        lax.fori_loop(0, NC, chunk_body, 0)

    return scat_kernel(yflat, src, dst, ew, dinv)


def _c1_body(z_ref, t_ref, out_ref, stats_ref, acc_ref):
    i = pl.program_id(0)
    nb = pl.num_programs(0)
    t = t_ref[...]
    NC = t.shape[1]
    o = z_ref[...] + jnp.concatenate(
        [t[0, cc] + t[1, cc] for cc in range(NC)], axis=1)
    out_ref[...] = o

    @pl.when(i == 0)
    def _():
        acc_ref[...] = jnp.zeros_like(acc_ref)

    s1 = jnp.sum(o, axis=0, keepdims=True)
    s2 = jnp.sum(o * o, axis=0, keepdims=True)
    acc_ref[...] += jnp.concatenate([s1, s2], axis=0)

    @pl.when(i == nb - 1)
    def _():
        stats_ref[...] = acc_ref[...]


def _run_c1(z, t, RB):
    N, F = z.shape
    NC = t.shape[1]
    CW = t.shape[3]
    grid = (N // RB,)
    return pl.pallas_call(
        _c1_body, grid=grid,
        in_specs=[
            pl.BlockSpec((RB, F), lambda i: (i, 0)),
            pl.BlockSpec((2, NC, RB, CW), lambda i: (0, 0, i, 0)),
        ],
        out_specs=[
            pl.BlockSpec((RB, F), lambda i: (i, 0)),
            pl.BlockSpec((2, F), lambda i: (0, 0)),
        ],
        out_shape=[
            jax.ShapeDtypeStruct((N, F), F32),
            jax.ShapeDtypeStruct((2, F), F32),
        ],
        scratch_shapes=[pltpu.VMEM((2, F), F32)],
    )(z, t)


def _c2_body(o_ref, st_ref, W0_ref, W1_ref, b_ref, y2_ref, z2_ref, *, n):
    st = st_ref[...]
    mu = st[0:1] / n
    var = st[1:2] / n - mu * mu
    h = _lrelu((o_ref[...] - mu) * lax.rsqrt(var + 1e-5))
    y2 = _dot(h, W1_ref[...])
    for cc in range(y2.shape[1] // 64):
        y2_ref[cc] = y2[:, cc * 64:(cc + 1) * 64]
    z2_ref[...] = _dot(h, W0_ref[...]) + b_ref[...]


def _run_c2(out1, stats, W0, W1, b, RB):
    N, F = out1.shape
    O = W0.shape[1]
    NC2 = O // 64
    grid = (N // RB,)
    return pl.pallas_call(
        functools.partial(_c2_body, n=float(N)), grid=grid,
        in_specs=[
            pl.BlockSpec((RB, F), lambda i: (i, 0)),
            pl.BlockSpec((2, F), lambda i: (0, 0)),
            pl.BlockSpec(W0.shape, lambda i: (0, 0)),
            pl.BlockSpec(W1.shape, lambda i: (0, 0)),
            pl.BlockSpec((1, O), lambda i: (0, 0)),
        ],
        out_specs=[
            pl.BlockSpec((NC2, RB, 64), lambda i: (0, i, 0)),
            pl.BlockSpec((RB, O), lambda i: (i, 0)),
        ],
        out_shape=[
            jax.ShapeDtypeStruct((NC2, N, 64), F32),
            jax.ShapeDtypeStruct((N, O), F32),
        ],
    )(out1, stats, W0, W1, b)


def _d2_body(o_ref, st_ref, out_ref, *, n):
    st = st_ref[...]
    mu = st[0:1] / n
    var = st[1:2] / n - mu * mu
    out_ref[...] = _lrelu((o_ref[...] - mu) * lax.rsqrt(var + 1e-5))


def _run_d2(out2, stats, RB):
    N, F = out2.shape
    grid = (N // RB,)
    return pl.pallas_call(
        functools.partial(_d2_body, n=float(N)), grid=grid,
        in_specs=[
            pl.BlockSpec((RB, F), lambda i: (i, 0)),
            pl.BlockSpec((2, F), lambda i: (0, 0)),
        ],
        out_specs=pl.BlockSpec((RB, F), lambda i: (i, 0)),
        out_shape=jax.ShapeDtypeStruct((N, F), F32),
    )(out2, stats)


def kernel(x, latent_vector, style_vector, edge_index, edge_attr, batch_size,
           nroi, W_fc1, b_fc1, Ws_fc1, bs_fc1, W_fc2, b_fc2, Ws_fc2, bs_fc2,
           W_fc3, b_fc3, Ws_fc3, bs_fc3, W_fc4, b_fc4, Ws_fc4, bs_fc4,
           W0_g1, W1_g1, b_g1, W0_g2, W1_g2, b_g2):
    N, IN = x.shape
    B, LAT = latent_vector.shape
    R = N // B
    E = edge_index.shape[1]
    F2 = W_fc4.shape[1]
    O = W0_g2.shape[1]
    src = edge_index[0]
    dst = edge_index[1]

    degp = _sc_deg(dst, edge_attr, N, E).reshape(2, N)

    Ws = [W_fc1, b_fc1.reshape(1, -1), Ws_fc1, bs_fc1.reshape(1, -1),
          W_fc2, b_fc2.reshape(1, -1), Ws_fc2, bs_fc2.reshape(1, -1),
          W_fc3, b_fc3.reshape(1, -1), Ws_fc3, bs_fc3.reshape(1, -1),
          W_fc4, b_fc4.reshape(1, -1), Ws_fc4, bs_fc4.reshape(1, -1),
          W0_g1, W1_g1, b_g1.reshape(1, -1)]
    y1, z1 = _run_a(x, latent_vector, style_vector, Ws, F2)
    dinv = _run_dinv(degp).reshape(N)

    Npad = -(-N // 2048) * 2048

    def unpad(t, NC):
        return t.reshape(2, NC, Npad, 64)[:, :, :N]

    NC1 = F2 // 64
    NC2 = O // 64
    t1 = unpad(_sc_edge_scatter(y1.reshape(NC1 * N, 64), src, dst,
                                edge_attr, dinv, N, E, NC1), NC1)
    out1, stats1 = _run_c1(z1, t1, 1000)
    y2, z2 = _run_c2(out1, stats1, W0_g2, W1_g2, b_g2.reshape(1, -1), 1000)

    t2 = unpad(_sc_edge_scatter(y2.reshape(NC2 * N, 64), src, dst,
                                edge_attr, dinv, N, E, NC2), NC2)
    out2, stats2 = _run_c1(z2, t2, 1000)
    h = _run_d2(out2, stats2, 1000)
    return h.reshape(B, R, O)
